# Initial kernel scaffold; baseline (speedup 1.0000x reference)
import jax
import jax.numpy as jnp
from jax.experimental import pallas as pl


def kernel(x, edge_attr, senders, receivers, We1, be1, We2, be2, Wn1, bn1, Wn2, bn2, bg_enc, We_e, We_s, We_g, be, Wn_n, Wn_i, Wn_g, bn, Wg_e, Wg_n, Wg_g, bg, Wr_n, br_n, Wr_g, br_g):
    raise NotImplementedError("write your pallas kernel here")


# speedup vs baseline: 1.8484x; 1.8484x over previous
import jax
import jax.numpy as jnp
from jax import lax
from jax.experimental import pallas as pl
from jax.experimental.pallas import tpu as pltpu
from jax.experimental.pallas import tpu_sc as plsc

N_NODES = 50000
N_EDGES = 800000
HALF = 32
PACK = 128 // HALF
EP_ROWS = N_EDGES // PACK
CHUNK = 128
CROWS = CHUNK // PACK
NCHUNKS = N_EDGES // CHUNK
HCHUNKS = NCHUNKS // 2
NS = 16
N_PAD = 50048
ROWS_PT = N_PAD // NS
EDGE_BLK = 1600
NODE_BLK = 1000
f32 = jnp.float32
i32 = jnp.int32


def _sigmoid(v):
    return 1.0 / (1.0 + jnp.exp(-v))


def _full(shape):
    n = len(shape)
    return pl.BlockSpec(shape, lambda i, _n=n: (0,) * _n)


def _edge_enc_body(eap, W1b, b1b, W2b, b2b, Web0, Web1, Wgb0, Wgb1, beb0,
                   beb1, bg4, ep0_o, ep1_o):
    u4 = jnp.maximum(bg4[...], 0.0)
    h = jnp.maximum(jnp.dot(eap[...], W1b[...], preferred_element_type=f32) + b1b[...], 0.0)
    h = jnp.maximum(jnp.dot(h, W2b[...], preferred_element_type=f32) + b2b[...], 0.0)
    ce0 = jnp.dot(u4, Wgb0[...], preferred_element_type=f32) + beb0[...]
    ce1 = jnp.dot(u4, Wgb1[...], preferred_element_type=f32) + beb1[...]
    ep0_o[...] = jnp.dot(h, Web0[...], preferred_element_type=f32) + ce0
    ep1_o[...] = jnp.dot(h, Web1[...], preferred_element_type=f32) + ce1


def _bd4(W):
    r, c = W.shape
    out = jnp.zeros((4 * r, 4 * c), W.dtype)
    for k in range(4):
        out = out.at[k * r:(k + 1) * r, k * c:(k + 1) * c].set(W)
    return out


def _edge_encode(edge_attr, We1, be1, We2, be2, We_e, We_g, be, bg_enc):
    eap = edge_attr.reshape(EP_ROWS, 2 * PACK)
    t4 = lambda v: jnp.tile(v.reshape(1, -1), (1, 4))
    grid = (EP_ROWS // (EDGE_BLK // PACK),)
    blk = EDGE_BLK // PACK
    return pl.pallas_call(
        _edge_enc_body,
        grid=grid,
        in_specs=[
            pl.BlockSpec((blk, 2 * PACK), lambda i: (i, 0)),
            _full((2 * PACK, 16)), _full((1, 16)), _full((16, 32)),
            _full((1, 32)), _full((32, 128)), _full((32, 128)),
            _full((16, 128)), _full((16, 128)), _full((1, 128)),
            _full((1, 128)), _full((1, 16)),
        ],
        out_specs=[pl.BlockSpec((blk, 128), lambda i: (i, 0))] * 2,
        out_shape=[jax.ShapeDtypeStruct((EP_ROWS, 128), f32)] * 2,
    )(eap, _bd4(We1), t4(be1), _bd4(We2), t4(be2), _bd4(We_e[:, :HALF]),
      _bd4(We_e[:, HALF:]), _bd4(We_g[:, :HALF]), _bd4(We_g[:, HALF:]),
      t4(be[:HALF]), t4(be[HALF:]), t4(bg_enc))


def _node_enc_body(x, Wn1, bn1, Wn2, bn2, Wes, n_o, np0_o, np1_o):
    h = jnp.maximum(jnp.dot(x[...], Wn1[...], preferred_element_type=f32) + bn1[...], 0.0)
    n = jnp.maximum(jnp.dot(h, Wn2[...], preferred_element_type=f32) + bn2[...], 0.0)
    npj = jnp.dot(n, Wes[...], preferred_element_type=f32)
    n_o[...] = n
    np0_o[...] = npj[:, :HALF]
    np1_o[...] = npj[:, HALF:]


def _node_encode(x, Wn1, bn1, Wn2, bn2, We_s):
    grid = (N_NODES // NODE_BLK,)
    return pl.pallas_call(
        _node_enc_body,
        grid=grid,
        in_specs=[
            pl.BlockSpec((NODE_BLK, 83), lambda i: (i, 0)),
            _full((83, 8)), _full((1, 8)), _full((8, 16)), _full((1, 16)),
            _full((16, 64)),
        ],
        out_specs=[pl.BlockSpec((NODE_BLK, 16), lambda i: (i, 0))]
        + [pl.BlockSpec((NODE_BLK, HALF), lambda i: (i, 0))] * 2,
        out_shape=[jax.ShapeDtypeStruct((N_NODES, 16), f32)]
        + [jax.ShapeDtypeStruct((N_PAD, HALF), f32)] * 2,
    )(x, Wn1, bn1, Wn2, bn2, We_s)


def _sc_body(ep0, ep1, np0, np1, snd, rcv, zh, ones_h,
             seg0_o, seg1_o, cnta_o, cntb_o,
             idx_s, idx_r, msg, gat, sct, ones_v, seg_sh, sem):
    s = lax.axis_index("s")
    c = lax.axis_index("c")
    r0 = s * ROWS_PT
    rows = pl.ds(r0, ROWS_PT)

    def do_pass(ep_h, np_h, seg_o):
        pltpu.sync_copy(zh.at[rows], seg_sh.at[rows])
        plsc.subcore_barrier()

        def chunk_body(j, carry):
            ck = j * NS + s

            @pl.when(ck < NCHUNKS)
            def _():
                off = ck * CHUNK
                pltpu.sync_copy(snd.at[pl.ds(off, CHUNK)], idx_s)
                pltpu.sync_copy(rcv.at[pl.ds(off, CHUNK)], idx_r)
                pltpu.sync_copy(ep_h.at[pl.ds(ck * CROWS, CROWS)], msg)
                pltpu.async_copy(np_h.at[idx_s], gat, sem).wait()

                def row_body(r, rc):
                    for e4 in range(PACK):
                        e = r * PACK + e4
                        for kk in range(HALF // 16):
                            src = pl.ds(e4 * HALF + kk * 16, 16)
                            dst = pl.ds(kk * 16, 16)
                            sct[e, dst] = jnp.maximum(msg[r, src] + gat[e, dst], 0.0)
                    return rc

                lax.fori_loop(0, CROWS, row_body, 0)
                pltpu.sync_copy(sct, seg_sh.at[idx_r], add=True)

            return carry

        lax.fori_loop(0, (NCHUNKS + NS - 1) // NS, chunk_body, 0)
        plsc.subcore_barrier()
        pltpu.sync_copy(seg_sh.at[rows], seg_o.at[rows])

    def do_cnt_pass(cnt_o, ck_base):
        pltpu.sync_copy(zh.at[rows], seg_sh.at[rows])
        pltpu.sync_copy(ones_h, ones_v)
        plsc.subcore_barrier()

        def chunk_body(j, carry):
            ck = ck_base + j * NS + s

            @pl.when(ck < ck_base + HCHUNKS)
            def _():
                pltpu.sync_copy(rcv.at[pl.ds(ck * CHUNK, CHUNK)], idx_r)
                pltpu.sync_copy(ones_v, seg_sh.at[idx_r], add=True)

            return carry

        lax.fori_loop(0, (HCHUNKS + NS - 1) // NS, chunk_body, 0)
        plsc.subcore_barrier()
        pltpu.sync_copy(seg_sh.at[rows], cnt_o.at[rows])

    @pl.when(c == 0)
    def _():
        do_pass(ep0, np0, seg0_o)
        do_cnt_pass(cnta_o, 0)

    @pl.when(c == 1)
    def _():
        do_pass(ep1, np1, seg1_o)
        do_cnt_pass(cntb_o, HCHUNKS)


def _sc_scatter(eps, nps, senders, receivers, zh, ones32):
    mesh = plsc.VectorSubcoreMesh(
        core_axis_name="c", subcore_axis_name="s", num_cores=2, num_subcores=NS)
    return pl.kernel(
        _sc_body,
        out_type=tuple([jax.ShapeDtypeStruct((N_PAD, HALF), f32)] * 4),
        mesh=mesh,
        compiler_params=pltpu.CompilerParams(use_tc_tiling_on_sc=False),
        scratch_types=[
            pltpu.VMEM((CHUNK,), i32),
            pltpu.VMEM((CHUNK,), i32),
            pltpu.VMEM((CROWS, 128), f32),
            pltpu.VMEM((CHUNK, HALF), f32),
            pltpu.VMEM((CHUNK, HALF), f32),
            pltpu.VMEM((CHUNK, HALF), f32),
            pltpu.VMEM_SHARED((N_PAD, HALF), f32),
            pltpu.SemaphoreType.DMA,
        ],
    )(*eps, *nps, senders, receivers, zh, ones32)


def _update_body(seg0, seg1, cnta, cntb, n, bg, Wnn, Wni, Wng, bn_, Wge,
                 Wgn, Wgg, bg_, Wrn, brn, Wrg, brg, nout_o, gout_o, acc_e, acc_n):
    i = pl.program_id(0)
    nblocks = pl.num_programs(0)

    @pl.when(i == 0)
    def _():
        acc_e[...] = jnp.zeros_like(acc_e)
        acc_n[...] = jnp.zeros_like(acc_n)

    seg = jnp.concatenate([seg0[...], seg1[...]], axis=1)
    deg = jnp.maximum(cnta[:, 0:1] + cntb[:, 0:1], 1.0)
    inc = seg / deg
    u = jnp.maximum(bg[...], 0.0)
    cn = jnp.dot(u, Wng[...], preferred_element_type=f32) + bn_[...]
    n2 = jnp.maximum(
        jnp.dot(n[...], Wnn[...], preferred_element_type=f32)
        + jnp.dot(inc, Wni[...], preferred_element_type=f32) + cn, 0.0)
    nout_o[...] = _sigmoid(jnp.dot(n2, Wrn[...], preferred_element_type=f32) + brn[...])
    acc_e[...] += jnp.sum(seg, axis=0, keepdims=True)
    acc_n[...] += jnp.sum(n2, axis=0, keepdims=True)

    @pl.when(i == nblocks - 1)
    def _():
        mean_e2 = acc_e[...] * (1.0 / N_EDGES)
        mean_n2 = acc_n[...] * (1.0 / N_NODES)
        u2 = jnp.maximum(
            jnp.dot(mean_e2, Wge[...], preferred_element_type=f32)
            + jnp.dot(mean_n2, Wgn[...], preferred_element_type=f32)
            + jnp.dot(u, Wgg[...], preferred_element_type=f32) + bg_[...], 0.0)
        gout_o[...] = _sigmoid(jnp.dot(u2, Wrg[...], preferred_element_type=f32) + brg[...])


def _update(seg0, seg1, cnta, cntb, n, bg, Wn_n, Wn_i, Wn_g, bn_, Wg_e, Wg_n,
            Wg_g, bg_, Wr_n, br_n, Wr_g, br_g):
    grid = (N_NODES // NODE_BLK,)
    return pl.pallas_call(
        _update_body,
        grid=grid,
        in_specs=[pl.BlockSpec((NODE_BLK, HALF), lambda i: (i, 0))] * 4 + [
            pl.BlockSpec((NODE_BLK, 16), lambda i: (i, 0)),
            _full((1, 4)), _full((16, 128)), _full((64, 128)), _full((4, 128)),
            _full((1, 128)), _full((64, 32)), _full((128, 32)), _full((4, 32)),
            _full((1, 32)), _full((128, 1)), _full((1, 1)), _full((32, 1)),
            _full((1, 1)),
        ],
        out_specs=[
            pl.BlockSpec((NODE_BLK, 1), lambda i: (i, 0)),
            pl.BlockSpec((1, 1), lambda i: (0, 0)),
        ],
        out_shape=[
            jax.ShapeDtypeStruct((N_NODES, 1), f32),
            jax.ShapeDtypeStruct((1, 1), f32),
        ],
        scratch_shapes=[
            pltpu.VMEM((1, 64), f32),
            pltpu.VMEM((1, 128), f32),
        ],
    )(seg0, seg1, cnta, cntb, n, bg, Wn_n, Wn_i, Wn_g, bn_, Wg_e, Wg_n,
      Wg_g, bg_, Wr_n, br_n, Wr_g, br_g)


def kernel(x, edge_attr, senders, receivers, We1, be1, We2, be2, Wn1, bn1,
           Wn2, bn2, bg_enc, We_e, We_s, We_g, be, Wn_n, Wn_i, Wn_g, bn,
           Wg_e, Wg_n, Wg_g, bg, Wr_n, br_n, Wr_g, br_g):
    r = lambda v: v.reshape(1, -1)
    eps = _edge_encode(edge_attr, We1, be1, We2, be2, We_e, We_g, be, bg_enc)
    n_enc, np0, np1 = _node_encode(x, Wn1, r(bn1), Wn2, r(bn2), We_s)
    zh = jnp.zeros((N_PAD, HALF), f32)
    ones32 = jnp.ones((CHUNK, HALF), f32)
    seg0, seg1, cnta, cntb = _sc_scatter(eps, (np0, np1), senders, receivers,
                                         zh, ones32)
    node_out, global_out = _update(seg0, seg1, cnta, cntb, n_enc, r(bg_enc),
                                   Wn_n, Wn_i, Wn_g, r(bn), Wg_e, Wg_n, Wg_g,
                                   r(bg), Wr_n, r(br_n), Wr_g, r(br_g))
    return node_out, global_out


# speedup vs baseline: 2.4255x; 1.3122x over previous
import jax
import jax.numpy as jnp
from jax import lax
from jax.experimental import pallas as pl
from jax.experimental.pallas import tpu as pltpu
from jax.experimental.pallas import tpu_sc as plsc

N_NODES = 50000
N_EDGES = 800000
HALF = 32
PACK = 128 // HALF
EP_ROWS = N_EDGES // PACK
CHUNK = 128
CROWS = CHUNK // PACK
NCHUNKS = N_EDGES // CHUNK
HCHUNKS = NCHUNKS // 2
NS = 16
N_PAD = 50048
ROWS_PT = N_PAD // NS
EDGE_BLK = 1600
NODE_BLK = 1000
f32 = jnp.float32
i32 = jnp.int32


def _sigmoid(v):
    return 1.0 / (1.0 + jnp.exp(-v))


def _full(shape):
    n = len(shape)
    return pl.BlockSpec(shape, lambda i, _n=n: (0,) * _n)


def _edge_enc_body(eap, W1b, b1b, W2b, b2b, Web0, Web1, Wgb0, Wgb1, beb0,
                   beb1, bg4, ep0_o, ep1_o):
    u4 = jnp.maximum(bg4[...], 0.0)
    h = jnp.maximum(jnp.dot(eap[...], W1b[...], preferred_element_type=f32) + b1b[...], 0.0)
    h = jnp.maximum(jnp.dot(h, W2b[...], preferred_element_type=f32) + b2b[...], 0.0)
    ce0 = jnp.dot(u4, Wgb0[...], preferred_element_type=f32) + beb0[...]
    ce1 = jnp.dot(u4, Wgb1[...], preferred_element_type=f32) + beb1[...]
    ep0_o[...] = jnp.dot(h, Web0[...], preferred_element_type=f32) + ce0
    ep1_o[...] = jnp.dot(h, Web1[...], preferred_element_type=f32) + ce1


def _bd4(W):
    r, c = W.shape
    out = jnp.zeros((4 * r, 4 * c), W.dtype)
    for k in range(4):
        out = out.at[k * r:(k + 1) * r, k * c:(k + 1) * c].set(W)
    return out


def _edge_encode(edge_attr, We1, be1, We2, be2, We_e, We_g, be, bg_enc):
    eap = edge_attr.reshape(EP_ROWS, 2 * PACK)
    t4 = lambda v: jnp.tile(v.reshape(1, -1), (1, 4))
    grid = (EP_ROWS // (EDGE_BLK // PACK),)
    blk = EDGE_BLK // PACK
    return pl.pallas_call(
        _edge_enc_body,
        grid=grid,
        in_specs=[
            pl.BlockSpec((blk, 2 * PACK), lambda i: (i, 0)),
            _full((2 * PACK, 16)), _full((1, 16)), _full((16, 32)),
            _full((1, 32)), _full((32, 128)), _full((32, 128)),
            _full((16, 128)), _full((16, 128)), _full((1, 128)),
            _full((1, 128)), _full((1, 16)),
        ],
        out_specs=[pl.BlockSpec((blk, 128), lambda i: (i, 0))] * 2,
        out_shape=[jax.ShapeDtypeStruct((EP_ROWS, 128), f32)] * 2,
    )(eap, _bd4(We1), t4(be1), _bd4(We2), t4(be2), _bd4(We_e[:, :HALF]),
      _bd4(We_e[:, HALF:]), _bd4(We_g[:, :HALF]), _bd4(We_g[:, HALF:]),
      t4(be[:HALF]), t4(be[HALF:]), t4(bg_enc))


def _node_enc_body(x, Wn1, bn1, Wn2, bn2, Wes, n_o, np0_o, np1_o):
    h = jnp.maximum(jnp.dot(x[...], Wn1[...], preferred_element_type=f32) + bn1[...], 0.0)
    n = jnp.maximum(jnp.dot(h, Wn2[...], preferred_element_type=f32) + bn2[...], 0.0)
    npj = jnp.dot(n, Wes[...], preferred_element_type=f32)
    n_o[...] = n
    np0_o[...] = npj[:, :HALF]
    np1_o[...] = npj[:, HALF:]


def _node_encode(x, Wn1, bn1, Wn2, bn2, We_s):
    grid = (N_NODES // NODE_BLK,)
    return pl.pallas_call(
        _node_enc_body,
        grid=grid,
        in_specs=[
            pl.BlockSpec((NODE_BLK, 83), lambda i: (i, 0)),
            _full((83, 8)), _full((1, 8)), _full((8, 16)), _full((1, 16)),
            _full((16, 64)),
        ],
        out_specs=[pl.BlockSpec((NODE_BLK, 16), lambda i: (i, 0))]
        + [pl.BlockSpec((NODE_BLK, HALF), lambda i: (i, 0))] * 2,
        out_shape=[jax.ShapeDtypeStruct((N_NODES, 16), f32)]
        + [jax.ShapeDtypeStruct((N_PAD, HALF), f32)] * 2,
    )(x, Wn1, bn1, Wn2, bn2, We_s)


NJT = (NCHUNKS + NS - 1) // NS
NJ2 = (NJT + 1) // 2
HJT = (HCHUNKS + NS - 1) // NS
HJ2 = (HJT + 1) // 2


def _sc_body(ep0, ep1, np0, np1, snd, rcv, zh, ones_h,
             seg0_o, seg1_o, cnta_o, cntb_o,
             idx_s0, idx_s1, idx_r0, idx_r1, msg0, msg1, gat0, gat1,
             sct0, sct1, ones_v, seg_sh, sem_l0, sem_l1, sem_g0, sem_g1):
    s = lax.axis_index("s")
    c = lax.axis_index("c")
    r0 = s * ROWS_PT
    rows = pl.ds(r0, ROWS_PT)
    base = s * NJT
    idx_s = (idx_s0, idx_s1)
    idx_r = (idx_r0, idx_r1)
    msg = (msg0, msg1)
    gat = (gat0, gat1)
    sct = (sct0, sct1)
    sem_l = (sem_l0, sem_l1)
    sem_g = (sem_g0, sem_g1)

    def do_pass(ep_h, np_h, seg_o):
        pltpu.sync_copy(zh.at[rows], seg_sh.at[rows])
        plsc.subcore_barrier()

        def valid(j):
            return jnp.logical_and(j < NJT, base + j < NCHUNKS)

        def fire_loads(j, b):
            @pl.when(valid(j))
            def _():
                ck = base + j
                off = ck * CHUNK
                pltpu.async_copy(snd.at[pl.ds(off, CHUNK)], idx_s[b], sem_l[b])
                pltpu.async_copy(rcv.at[pl.ds(off, CHUNK)], idx_r[b], sem_l[b])
                pltpu.async_copy(ep_h.at[pl.ds(ck * CROWS, CROWS)], msg[b], sem_l[b])

        def wait_loads(j, b):
            @pl.when(valid(j))
            def _():
                ck = base + j
                off = ck * CHUNK
                pltpu.make_async_copy(snd.at[pl.ds(off, CHUNK)], idx_s[b], sem_l[b]).wait()
                pltpu.make_async_copy(rcv.at[pl.ds(off, CHUNK)], idx_r[b], sem_l[b]).wait()
                pltpu.make_async_copy(ep_h.at[pl.ds(0, CROWS)], msg[b], sem_l[b]).wait()

        def fire_gather(j, b):
            @pl.when(valid(j))
            def _():
                pltpu.async_copy(np_h.at[idx_s[b]], gat[b], sem_g[b])

        def process(j, b):
            @pl.when(valid(j))
            def _():
                pltpu.make_async_copy(np_h.at[idx_s[b]], gat[b], sem_g[b]).wait()

                def row_body(r, rc):
                    for e4 in range(PACK):
                        e = r * PACK + e4
                        for kk in range(HALF // 16):
                            src = pl.ds(e4 * HALF + kk * 16, 16)
                            dst = pl.ds(kk * 16, 16)
                            sct[b][e, dst] = jnp.maximum(
                                msg[b][r, src] + gat[b][e, dst], 0.0)
                    return rc

                lax.fori_loop(0, CROWS, row_body, 0)
                pltpu.sync_copy(sct[b], seg_sh.at[idx_r[b]], add=True)

        fire_loads(0, 0)
        fire_loads(1, 1)
        wait_loads(0, 0)
        fire_gather(0, 0)

        def loop(jo, carry):
            for b in range(2):
                j = jo * 2 + b
                process(j, b)
                fire_loads(j + 2, b)
                wait_loads(j + 1, 1 - b)
                fire_gather(j + 1, 1 - b)
            return carry

        lax.fori_loop(0, NJ2, loop, 0)
        plsc.subcore_barrier()
        pltpu.sync_copy(seg_sh.at[rows], seg_o.at[rows])

    def do_cnt_pass(cnt_o, ck_base):
        pltpu.sync_copy(zh.at[rows], seg_sh.at[rows])
        pltpu.sync_copy(ones_h, ones_v)
        plsc.subcore_barrier()
        cbase = ck_base + s * HJT

        def valid(j):
            return jnp.logical_and(j < HJT, cbase + j < ck_base + HCHUNKS)

        def fire(j, b):
            @pl.when(valid(j))
            def _():
                off = (cbase + j) * CHUNK
                pltpu.async_copy(rcv.at[pl.ds(off, CHUNK)], idx_r[b], sem_l[b])

        def process(j, b):
            @pl.when(valid(j))
            def _():
                off = (cbase + j) * CHUNK
                pltpu.make_async_copy(rcv.at[pl.ds(off, CHUNK)], idx_r[b], sem_l[b]).wait()
                pltpu.sync_copy(ones_v, seg_sh.at[idx_r[b]], add=True)

        fire(0, 0)
        fire(1, 1)

        def loop(jo, carry):
            for b in range(2):
                j = jo * 2 + b
                process(j, b)
                fire(j + 2, b)
            return carry

        lax.fori_loop(0, HJ2, loop, 0)
        plsc.subcore_barrier()
        pltpu.sync_copy(seg_sh.at[rows], cnt_o.at[rows])

    @pl.when(c == 0)
    def _():
        do_pass(ep0, np0, seg0_o)
        do_cnt_pass(cnta_o, 0)

    @pl.when(c == 1)
    def _():
        do_pass(ep1, np1, seg1_o)
        do_cnt_pass(cntb_o, HCHUNKS)


def _sc_scatter(eps, nps, senders, receivers, zh, ones32):
    mesh = plsc.VectorSubcoreMesh(
        core_axis_name="c", subcore_axis_name="s", num_cores=2, num_subcores=NS)
    return pl.kernel(
        _sc_body,
        out_type=tuple([jax.ShapeDtypeStruct((N_PAD, HALF), f32)] * 4),
        mesh=mesh,
        compiler_params=pltpu.CompilerParams(use_tc_tiling_on_sc=False),
        scratch_types=[
            pltpu.VMEM((CHUNK,), i32),
            pltpu.VMEM((CHUNK,), i32),
            pltpu.VMEM((CHUNK,), i32),
            pltpu.VMEM((CHUNK,), i32),
            pltpu.VMEM((CROWS, 128), f32),
            pltpu.VMEM((CROWS, 128), f32),
            pltpu.VMEM((CHUNK, HALF), f32),
            pltpu.VMEM((CHUNK, HALF), f32),
            pltpu.VMEM((CHUNK, HALF), f32),
            pltpu.VMEM((CHUNK, HALF), f32),
            pltpu.VMEM((CHUNK, HALF), f32),
            pltpu.VMEM_SHARED((N_PAD, HALF), f32),
            pltpu.SemaphoreType.DMA,
            pltpu.SemaphoreType.DMA,
            pltpu.SemaphoreType.DMA,
            pltpu.SemaphoreType.DMA,
        ],
    )(*eps, *nps, senders, receivers, zh, ones32)


def _update_body(seg0, seg1, cnta, cntb, n, bg, Wnn, Wni, Wng, bn_, Wge,
                 Wgn, Wgg, bg_, Wrn, brn, Wrg, brg, nout_o, gout_o, acc_e, acc_n):
    i = pl.program_id(0)
    nblocks = pl.num_programs(0)

    @pl.when(i == 0)
    def _():
        acc_e[...] = jnp.zeros_like(acc_e)
        acc_n[...] = jnp.zeros_like(acc_n)

    seg = jnp.concatenate([seg0[...], seg1[...]], axis=1)
    deg = jnp.maximum(cnta[:, 0:1] + cntb[:, 0:1], 1.0)
    inc = seg / deg
    u = jnp.maximum(bg[...], 0.0)
    cn = jnp.dot(u, Wng[...], preferred_element_type=f32) + bn_[...]
    n2 = jnp.maximum(
        jnp.dot(n[...], Wnn[...], preferred_element_type=f32)
        + jnp.dot(inc, Wni[...], preferred_element_type=f32) + cn, 0.0)
    nout_o[...] = _sigmoid(jnp.dot(n2, Wrn[...], preferred_element_type=f32) + brn[...])
    acc_e[...] += jnp.sum(seg, axis=0, keepdims=True)
    acc_n[...] += jnp.sum(n2, axis=0, keepdims=True)

    @pl.when(i == nblocks - 1)
    def _():
        mean_e2 = acc_e[...] * (1.0 / N_EDGES)
        mean_n2 = acc_n[...] * (1.0 / N_NODES)
        u2 = jnp.maximum(
            jnp.dot(mean_e2, Wge[...], preferred_element_type=f32)
            + jnp.dot(mean_n2, Wgn[...], preferred_element_type=f32)
            + jnp.dot(u, Wgg[...], preferred_element_type=f32) + bg_[...], 0.0)
        gout_o[...] = _sigmoid(jnp.dot(u2, Wrg[...], preferred_element_type=f32) + brg[...])


def _update(seg0, seg1, cnta, cntb, n, bg, Wn_n, Wn_i, Wn_g, bn_, Wg_e, Wg_n,
            Wg_g, bg_, Wr_n, br_n, Wr_g, br_g):
    grid = (N_NODES // NODE_BLK,)
    return pl.pallas_call(
        _update_body,
        grid=grid,
        in_specs=[pl.BlockSpec((NODE_BLK, HALF), lambda i: (i, 0))] * 4 + [
            pl.BlockSpec((NODE_BLK, 16), lambda i: (i, 0)),
            _full((1, 4)), _full((16, 128)), _full((64, 128)), _full((4, 128)),
            _full((1, 128)), _full((64, 32)), _full((128, 32)), _full((4, 32)),
            _full((1, 32)), _full((128, 1)), _full((1, 1)), _full((32, 1)),
            _full((1, 1)),
        ],
        out_specs=[
            pl.BlockSpec((NODE_BLK, 1), lambda i: (i, 0)),
            pl.BlockSpec((1, 1), lambda i: (0, 0)),
        ],
        out_shape=[
            jax.ShapeDtypeStruct((N_NODES, 1), f32),
            jax.ShapeDtypeStruct((1, 1), f32),
        ],
        scratch_shapes=[
            pltpu.VMEM((1, 64), f32),
            pltpu.VMEM((1, 128), f32),
        ],
    )(seg0, seg1, cnta, cntb, n, bg, Wn_n, Wn_i, Wn_g, bn_, Wg_e, Wg_n,
      Wg_g, bg_, Wr_n, br_n, Wr_g, br_g)


def kernel(x, edge_attr, senders, receivers, We1, be1, We2, be2, Wn1, bn1,
           Wn2, bn2, bg_enc, We_e, We_s, We_g, be, Wn_n, Wn_i, Wn_g, bn,
           Wg_e, Wg_n, Wg_g, bg, Wr_n, br_n, Wr_g, br_g):
    r = lambda v: v.reshape(1, -1)
    eps = _edge_encode(edge_attr, We1, be1, We2, be2, We_e, We_g, be, bg_enc)
    n_enc, np0, np1 = _node_encode(x, Wn1, r(bn1), Wn2, r(bn2), We_s)
    zh = jnp.zeros((N_PAD, HALF), f32)
    ones32 = jnp.ones((CHUNK, HALF), f32)
    seg0, seg1, cnta, cntb = _sc_scatter(eps, (np0, np1), senders, receivers,
                                         zh, ones32)
    node_out, global_out = _update(seg0, seg1, cnta, cntb, n_enc, r(bg_enc),
                                   Wn_n, Wn_i, Wn_g, r(bn), Wg_e, Wg_n, Wg_g,
                                   r(bg), Wr_n, r(br_n), Wr_g, r(br_g))
    return node_out, global_out


# speedup vs baseline: 2.4302x; 1.0019x over previous
import jax
import jax.numpy as jnp
from jax import lax
from jax.experimental import pallas as pl
from jax.experimental.pallas import tpu as pltpu
from jax.experimental.pallas import tpu_sc as plsc

N_NODES = 50000
N_EDGES = 800000
HALF = 32
PACK = 128 // HALF
EP_ROWS = N_EDGES // PACK
CHUNK = 128
CROWS = CHUNK // PACK
NCHUNKS = N_EDGES // CHUNK
HCHUNKS = NCHUNKS // 2
NS = 16
N_PAD = 50048
ROWS_PT = N_PAD // NS
EDGE_BLK = 1600
NODE_BLK = 1000
f32 = jnp.float32
i32 = jnp.int32


def _sigmoid(v):
    return 1.0 / (1.0 + jnp.exp(-v))


def _full(shape):
    n = len(shape)
    return pl.BlockSpec(shape, lambda i, _n=n: (0,) * _n)


def _edge_enc_body(eap, W1b, b1b, W2b, b2b, Web0, Web1, Wgb0, Wgb1, beb0,
                   beb1, bg4, ep0_o, ep1_o):
    u4 = jnp.maximum(bg4[...], 0.0)
    h = jnp.maximum(jnp.dot(eap[...], W1b[...], preferred_element_type=f32) + b1b[...], 0.0)
    h = jnp.maximum(jnp.dot(h, W2b[...], preferred_element_type=f32) + b2b[...], 0.0)
    ce0 = jnp.dot(u4, Wgb0[...], preferred_element_type=f32) + beb0[...]
    ce1 = jnp.dot(u4, Wgb1[...], preferred_element_type=f32) + beb1[...]
    ep0_o[...] = jnp.dot(h, Web0[...], preferred_element_type=f32) + ce0
    ep1_o[...] = jnp.dot(h, Web1[...], preferred_element_type=f32) + ce1


def _bd4(W):
    r, c = W.shape
    out = jnp.zeros((4 * r, 4 * c), W.dtype)
    for k in range(4):
        out = out.at[k * r:(k + 1) * r, k * c:(k + 1) * c].set(W)
    return out


def _edge_encode(edge_attr, We1, be1, We2, be2, We_e, We_g, be, bg_enc):
    eap = edge_attr.reshape(EP_ROWS, 2 * PACK)
    t4 = lambda v: jnp.tile(v.reshape(1, -1), (1, 4))
    grid = (EP_ROWS // (EDGE_BLK // PACK),)
    blk = EDGE_BLK // PACK
    return pl.pallas_call(
        _edge_enc_body,
        grid=grid,
        in_specs=[
            pl.BlockSpec((blk, 2 * PACK), lambda i: (i, 0)),
            _full((2 * PACK, 16)), _full((1, 16)), _full((16, 32)),
            _full((1, 32)), _full((32, 128)), _full((32, 128)),
            _full((16, 128)), _full((16, 128)), _full((1, 128)),
            _full((1, 128)), _full((1, 16)),
        ],
        out_specs=[pl.BlockSpec((blk, 128), lambda i: (i, 0))] * 2,
        out_shape=[jax.ShapeDtypeStruct((EP_ROWS, 128), f32)] * 2,
    )(eap, _bd4(We1), t4(be1), _bd4(We2), t4(be2), _bd4(We_e[:, :HALF]),
      _bd4(We_e[:, HALF:]), _bd4(We_g[:, :HALF]), _bd4(We_g[:, HALF:]),
      t4(be[:HALF]), t4(be[HALF:]), t4(bg_enc))


def _node_enc_body(x, Wn1, bn1, Wn2, bn2, Wes, n_o, np0_o, np1_o):
    h = jnp.maximum(jnp.dot(x[...], Wn1[...], preferred_element_type=f32) + bn1[...], 0.0)
    n = jnp.maximum(jnp.dot(h, Wn2[...], preferred_element_type=f32) + bn2[...], 0.0)
    npj = jnp.dot(n, Wes[...], preferred_element_type=f32)
    n_o[...] = n
    np0_o[...] = npj[:, :HALF]
    np1_o[...] = npj[:, HALF:]


def _node_encode(x, Wn1, bn1, Wn2, bn2, We_s):
    grid = (N_NODES // NODE_BLK,)
    return pl.pallas_call(
        _node_enc_body,
        grid=grid,
        in_specs=[
            pl.BlockSpec((NODE_BLK, 83), lambda i: (i, 0)),
            _full((83, 8)), _full((1, 8)), _full((8, 16)), _full((1, 16)),
            _full((16, 64)),
        ],
        out_specs=[pl.BlockSpec((NODE_BLK, 16), lambda i: (i, 0))]
        + [pl.BlockSpec((NODE_BLK, HALF), lambda i: (i, 0))] * 2,
        out_shape=[jax.ShapeDtypeStruct((N_NODES, 16), f32)]
        + [jax.ShapeDtypeStruct((N_PAD, HALF), f32)] * 2,
    )(x, Wn1, bn1, Wn2, bn2, We_s)


NJT = (NCHUNKS + NS - 1) // NS
NJ2 = (NJT + 1) // 2
HJT = (HCHUNKS + NS - 1) // NS
HJ2 = (HJT + 1) // 2


def _sc_body(ep0, ep1, np0, np1, snd, rcv, zh, ones_h,
             seg0_o, seg1_o, cnta_o, cntb_o,
             idx_s0, idx_s1, idx_r0, idx_r1, msg0, msg1, gat0, gat1,
             sct0, sct1, ones_v, seg_sh, sem_l0, sem_l1, sem_g0, sem_g1):
    s = lax.axis_index("s")
    c = lax.axis_index("c")
    r0 = s * ROWS_PT
    rows = pl.ds(r0, ROWS_PT)
    base = s * NJT
    idx_s = (idx_s0, idx_s1)
    idx_r = (idx_r0, idx_r1)
    msg = (msg0, msg1)
    gat = (gat0, gat1)
    sct = (sct0, sct1)
    sem_l = (sem_l0, sem_l1)
    sem_g = (sem_g0, sem_g1)

    def do_pass(ep_h, np_h, seg_o):
        pltpu.sync_copy(zh.at[rows], seg_sh.at[rows])
        plsc.subcore_barrier()

        def valid(j):
            return jnp.logical_and(j < NJT, base + j < NCHUNKS)

        def fire_loads(j, b):
            @pl.when(valid(j))
            def _():
                ck = base + j
                off = ck * CHUNK
                pltpu.async_copy(snd.at[pl.ds(off, CHUNK)], idx_s[b], sem_l[b])
                pltpu.async_copy(rcv.at[pl.ds(off, CHUNK)], idx_r[b], sem_l[b])
                pltpu.async_copy(ep_h.at[pl.ds(ck * CHUNK * HALF, CHUNK * HALF)], msg[b], sem_l[b])

        def wait_loads(j, b):
            @pl.when(valid(j))
            def _():
                ck = base + j
                off = ck * CHUNK
                pltpu.make_async_copy(snd.at[pl.ds(off, CHUNK)], idx_s[b], sem_l[b]).wait()
                pltpu.make_async_copy(rcv.at[pl.ds(off, CHUNK)], idx_r[b], sem_l[b]).wait()
                pltpu.make_async_copy(ep_h.at[pl.ds(0, CHUNK * HALF)], msg[b], sem_l[b]).wait()

        def fire_gather(j, b):
            @pl.when(valid(j))
            def _():
                pltpu.async_copy(np_h.at[idx_s[b]], gat[b], sem_g[b])

        def process(j, b):
            @pl.when(valid(j))
            def _():
                pltpu.make_async_copy(np_h.at[idx_s[b]], gat[b], sem_g[b]).wait()

                def row_body(r, rc):
                    for e4 in range(PACK):
                        e = r * PACK + e4
                        for kk in range(HALF // 16):
                            src = pl.ds(r * 128 + e4 * HALF + kk * 16, 16)
                            dst = pl.ds(kk * 16, 16)
                            sct[b][e, dst] = jnp.maximum(
                                msg[b][src] + gat[b][e, dst], 0.0)
                    return rc

                lax.fori_loop(0, CROWS, row_body, 0)
                pltpu.sync_copy(sct[b], seg_sh.at[idx_r[b]], add=True)

        fire_loads(0, 0)
        fire_loads(1, 1)
        wait_loads(0, 0)
        fire_gather(0, 0)

        def loop(jo, carry):
            for b in range(2):
                j = jo * 2 + b
                process(j, b)
                fire_loads(j + 2, b)
                wait_loads(j + 1, 1 - b)
                fire_gather(j + 1, 1 - b)
            return carry

        lax.fori_loop(0, NJ2, loop, 0)
        plsc.subcore_barrier()
        pltpu.sync_copy(seg_sh.at[rows], seg_o.at[rows])

    def do_cnt_pass(cnt_o, ck_base):
        pltpu.sync_copy(zh.at[rows], seg_sh.at[rows])
        pltpu.sync_copy(ones_h, ones_v)
        plsc.subcore_barrier()
        cbase = ck_base + s * HJT

        def valid(j):
            return jnp.logical_and(j < HJT, cbase + j < ck_base + HCHUNKS)

        def fire(j, b):
            @pl.when(valid(j))
            def _():
                off = (cbase + j) * CHUNK
                pltpu.async_copy(rcv.at[pl.ds(off, CHUNK)], idx_r[b], sem_l[b])

        def process(j, b):
            @pl.when(valid(j))
            def _():
                off = (cbase + j) * CHUNK
                pltpu.make_async_copy(rcv.at[pl.ds(off, CHUNK)], idx_r[b], sem_l[b]).wait()
                pltpu.sync_copy(ones_v, seg_sh.at[idx_r[b]], add=True)

        fire(0, 0)
        fire(1, 1)

        def loop(jo, carry):
            for b in range(2):
                j = jo * 2 + b
                process(j, b)
                fire(j + 2, b)
            return carry

        lax.fori_loop(0, HJ2, loop, 0)
        plsc.subcore_barrier()
        pltpu.sync_copy(seg_sh.at[rows], cnt_o.at[rows])

    @pl.when(c == 0)
    def _():
        do_pass(ep0, np0, seg0_o)
        do_cnt_pass(cnta_o, 0)

    @pl.when(c == 1)
    def _():
        do_pass(ep1, np1, seg1_o)
        do_cnt_pass(cntb_o, HCHUNKS)


def _sc_scatter(eps, nps, senders, receivers, zh, ones32):
    mesh = plsc.VectorSubcoreMesh(
        core_axis_name="c", subcore_axis_name="s", num_cores=2, num_subcores=NS)
    return pl.kernel(
        _sc_body,
        out_type=tuple([jax.ShapeDtypeStruct((N_PAD, HALF), f32)] * 4),
        mesh=mesh,
        compiler_params=pltpu.CompilerParams(use_tc_tiling_on_sc=False),
        scratch_types=[
            pltpu.VMEM((CHUNK,), i32),
            pltpu.VMEM((CHUNK,), i32),
            pltpu.VMEM((CHUNK,), i32),
            pltpu.VMEM((CHUNK,), i32),
            pltpu.VMEM((CHUNK * HALF,), f32),
            pltpu.VMEM((CHUNK * HALF,), f32),
            pltpu.VMEM((CHUNK, HALF), f32),
            pltpu.VMEM((CHUNK, HALF), f32),
            pltpu.VMEM((CHUNK, HALF), f32),
            pltpu.VMEM((CHUNK, HALF), f32),
            pltpu.VMEM((CHUNK, HALF), f32),
            pltpu.VMEM_SHARED((N_PAD, HALF), f32),
            pltpu.SemaphoreType.DMA,
            pltpu.SemaphoreType.DMA,
            pltpu.SemaphoreType.DMA,
            pltpu.SemaphoreType.DMA,
        ],
    )(*eps, *nps, senders, receivers, zh, ones32)


def _update_body(seg0, seg1, cnta, cntb, n, bg, Wnn, Wni, Wng, bn_, Wge,
                 Wgn, Wgg, bg_, Wrn, brn, Wrg, brg, nout_o, gout_o, acc_e, acc_n):
    i = pl.program_id(0)
    nblocks = pl.num_programs(0)

    @pl.when(i == 0)
    def _():
        acc_e[...] = jnp.zeros_like(acc_e)
        acc_n[...] = jnp.zeros_like(acc_n)

    seg = jnp.concatenate([seg0[...], seg1[...]], axis=1)
    deg = jnp.maximum(cnta[:, 0:1] + cntb[:, 0:1], 1.0)
    inc = seg / deg
    u = jnp.maximum(bg[...], 0.0)
    cn = jnp.dot(u, Wng[...], preferred_element_type=f32) + bn_[...]
    n2 = jnp.maximum(
        jnp.dot(n[...], Wnn[...], preferred_element_type=f32)
        + jnp.dot(inc, Wni[...], preferred_element_type=f32) + cn, 0.0)
    nout_o[...] = _sigmoid(jnp.dot(n2, Wrn[...], preferred_element_type=f32) + brn[...])
    acc_e[...] += jnp.sum(seg, axis=0, keepdims=True)
    acc_n[...] += jnp.sum(n2, axis=0, keepdims=True)

    @pl.when(i == nblocks - 1)
    def _():
        mean_e2 = acc_e[...] * (1.0 / N_EDGES)
        mean_n2 = acc_n[...] * (1.0 / N_NODES)
        u2 = jnp.maximum(
            jnp.dot(mean_e2, Wge[...], preferred_element_type=f32)
            + jnp.dot(mean_n2, Wgn[...], preferred_element_type=f32)
            + jnp.dot(u, Wgg[...], preferred_element_type=f32) + bg_[...], 0.0)
        gout_o[...] = _sigmoid(jnp.dot(u2, Wrg[...], preferred_element_type=f32) + brg[...])


def _update(seg0, seg1, cnta, cntb, n, bg, Wn_n, Wn_i, Wn_g, bn_, Wg_e, Wg_n,
            Wg_g, bg_, Wr_n, br_n, Wr_g, br_g):
    grid = (N_NODES // NODE_BLK,)
    return pl.pallas_call(
        _update_body,
        grid=grid,
        in_specs=[pl.BlockSpec((NODE_BLK, HALF), lambda i: (i, 0))] * 4 + [
            pl.BlockSpec((NODE_BLK, 16), lambda i: (i, 0)),
            _full((1, 4)), _full((16, 128)), _full((64, 128)), _full((4, 128)),
            _full((1, 128)), _full((64, 32)), _full((128, 32)), _full((4, 32)),
            _full((1, 32)), _full((128, 1)), _full((1, 1)), _full((32, 1)),
            _full((1, 1)),
        ],
        out_specs=[
            pl.BlockSpec((NODE_BLK, 1), lambda i: (i, 0)),
            pl.BlockSpec((1, 1), lambda i: (0, 0)),
        ],
        out_shape=[
            jax.ShapeDtypeStruct((N_NODES, 1), f32),
            jax.ShapeDtypeStruct((1, 1), f32),
        ],
        scratch_shapes=[
            pltpu.VMEM((1, 64), f32),
            pltpu.VMEM((1, 128), f32),
        ],
    )(seg0, seg1, cnta, cntb, n, bg, Wn_n, Wn_i, Wn_g, bn_, Wg_e, Wg_n,
      Wg_g, bg_, Wr_n, br_n, Wr_g, br_g)


def kernel(x, edge_attr, senders, receivers, We1, be1, We2, be2, Wn1, bn1,
           Wn2, bn2, bg_enc, We_e, We_s, We_g, be, Wn_n, Wn_i, Wn_g, bn,
           Wg_e, Wg_n, Wg_g, bg, Wr_n, br_n, Wr_g, br_g):
    r = lambda v: v.reshape(1, -1)
    eps = _edge_encode(edge_attr, We1, be1, We2, be2, We_e, We_g, be, bg_enc)
    n_enc, np0, np1 = _node_encode(x, Wn1, r(bn1), Wn2, r(bn2), We_s)
    zh = jnp.zeros((N_PAD, HALF), f32)
    ones32 = jnp.ones((CHUNK, HALF), f32)
    eps_flat = tuple(e.reshape(-1) for e in eps)
    seg0, seg1, cnta, cntb = _sc_scatter(eps_flat, (np0, np1), senders,
                                         receivers, zh, ones32)
    node_out, global_out = _update(seg0, seg1, cnta, cntb, n_enc, r(bg_enc),
                                   Wn_n, Wn_i, Wn_g, r(bn), Wg_e, Wg_n, Wg_g,
                                   r(bg), Wr_n, r(br_n), Wr_g, r(br_g))
    return node_out, global_out


# speedup vs baseline: 2.4362x; 1.0025x over previous
import jax
import jax.numpy as jnp
from jax import lax
from jax.experimental import pallas as pl
from jax.experimental.pallas import tpu as pltpu
from jax.experimental.pallas import tpu_sc as plsc

N_NODES = 50000
N_EDGES = 800000
HALF = 32
PACK = 128 // HALF
EP_ROWS = N_EDGES // PACK
CHUNK = 128
CROWS = CHUNK // PACK
NCHUNKS = N_EDGES // CHUNK
HCHUNKS = NCHUNKS // 2
NS = 16
N_PAD = 50048
ROWS_PT = N_PAD // NS
EDGE_BLK = 1600
NODE_BLK = 1000
f32 = jnp.float32
i32 = jnp.int32


def _sigmoid(v):
    return 1.0 / (1.0 + jnp.exp(-v))


def _full(shape):
    n = len(shape)
    return pl.BlockSpec(shape, lambda i, _n=n: (0,) * _n)


def _edge_enc_body(eap, W1b, b1b, W2b, b2b, Web0, Web1, Wgb0, Wgb1, beb0,
                   beb1, bg4, ep0_o, ep1_o):
    u4 = jnp.maximum(bg4[...], 0.0)
    h = jnp.maximum(jnp.dot(eap[...], W1b[...], preferred_element_type=f32) + b1b[...], 0.0)
    h = jnp.maximum(jnp.dot(h, W2b[...], preferred_element_type=f32) + b2b[...], 0.0)
    ce0 = jnp.dot(u4, Wgb0[...], preferred_element_type=f32) + beb0[...]
    ce1 = jnp.dot(u4, Wgb1[...], preferred_element_type=f32) + beb1[...]
    ep0_o[...] = jnp.dot(h, Web0[...], preferred_element_type=f32) + ce0
    ep1_o[...] = jnp.dot(h, Web1[...], preferred_element_type=f32) + ce1


def _bd4(W):
    r, c = W.shape
    out = jnp.zeros((4 * r, 4 * c), W.dtype)
    for k in range(4):
        out = out.at[k * r:(k + 1) * r, k * c:(k + 1) * c].set(W)
    return out


def _edge_encode(edge_attr, We1, be1, We2, be2, We_e, We_g, be, bg_enc):
    eap = (edge_attr + jnp.float32(1e-30)).reshape(EP_ROWS, 2 * PACK)
    t4 = lambda v: jnp.tile(v.reshape(1, -1), (1, 4))
    grid = (EP_ROWS // (EDGE_BLK // PACK),)
    blk = EDGE_BLK // PACK
    return pl.pallas_call(
        _edge_enc_body,
        grid=grid,
        in_specs=[
            pl.BlockSpec((blk, 2 * PACK), lambda i: (i, 0)),
            _full((2 * PACK, 16)), _full((1, 16)), _full((16, 32)),
            _full((1, 32)), _full((32, 128)), _full((32, 128)),
            _full((16, 128)), _full((16, 128)), _full((1, 128)),
            _full((1, 128)), _full((1, 16)),
        ],
        out_specs=[pl.BlockSpec((blk, 128), lambda i: (i, 0))] * 2,
        out_shape=[jax.ShapeDtypeStruct((EP_ROWS, 128), f32)] * 2,
    )(eap, _bd4(We1), t4(be1), _bd4(We2), t4(be2), _bd4(We_e[:, :HALF]),
      _bd4(We_e[:, HALF:]), _bd4(We_g[:, :HALF]), _bd4(We_g[:, HALF:]),
      t4(be[:HALF]), t4(be[HALF:]), t4(bg_enc))


def _node_enc_body(x, Wn1, bn1, Wn2, bn2, Wes, n_o, np0_o, np1_o):
    h = jnp.maximum(jnp.dot(x[...], Wn1[...], preferred_element_type=f32) + bn1[...], 0.0)
    n = jnp.maximum(jnp.dot(h, Wn2[...], preferred_element_type=f32) + bn2[...], 0.0)
    npj = jnp.dot(n, Wes[...], preferred_element_type=f32)
    n_o[...] = n
    np0_o[...] = npj[:, :HALF]
    np1_o[...] = npj[:, HALF:]


def _node_encode(x, Wn1, bn1, Wn2, bn2, We_s):
    grid = (N_NODES // NODE_BLK,)
    return pl.pallas_call(
        _node_enc_body,
        grid=grid,
        in_specs=[
            pl.BlockSpec((NODE_BLK, 83), lambda i: (i, 0)),
            _full((83, 8)), _full((1, 8)), _full((8, 16)), _full((1, 16)),
            _full((16, 64)),
        ],
        out_specs=[pl.BlockSpec((NODE_BLK, 16), lambda i: (i, 0))]
        + [pl.BlockSpec((NODE_BLK, HALF), lambda i: (i, 0))] * 2,
        out_shape=[jax.ShapeDtypeStruct((N_NODES, 16), f32)]
        + [jax.ShapeDtypeStruct((N_PAD, HALF), f32)] * 2,
    )(x, Wn1, bn1, Wn2, bn2, We_s)


NJT = (NCHUNKS + NS - 1) // NS
NJ2 = (NJT + 1) // 2
HJT = (HCHUNKS + NS - 1) // NS
HJ2 = (HJT + 1) // 2


def _sc_body(ep0, ep1, np0, np1, snd, rcv, zh, ones_h,
             seg0_o, seg1_o, cnta_o, cntb_o,
             idx_s0, idx_s1, idx_r0, idx_r1, msg0, msg1, gat0, gat1,
             sct0, sct1, ones_v, seg_sh, sem_l0, sem_l1, sem_g0, sem_g1):
    s = lax.axis_index("s")
    c = lax.axis_index("c")
    r0 = s * ROWS_PT
    rows = pl.ds(r0, ROWS_PT)
    base = s * NJT
    idx_s = (idx_s0, idx_s1)
    idx_r = (idx_r0, idx_r1)
    msg = (msg0, msg1)
    gat = (gat0, gat1)
    sct = (sct0, sct1)
    sem_l = (sem_l0, sem_l1)
    sem_g = (sem_g0, sem_g1)

    def do_pass(ep_h, np_h, seg_o):
        pltpu.sync_copy(zh.at[rows], seg_sh.at[rows])
        plsc.subcore_barrier()

        def valid(j):
            return jnp.logical_and(j < NJT, base + j < NCHUNKS)

        def fire_loads(j, b):
            @pl.when(valid(j))
            def _():
                ck = base + j
                off = ck * CHUNK
                pltpu.async_copy(snd.at[pl.ds(off, CHUNK)], idx_s[b], sem_l[b])
                pltpu.async_copy(rcv.at[pl.ds(off, CHUNK)], idx_r[b], sem_l[b])
                pltpu.async_copy(ep_h.at[pl.ds(ck * CHUNK * HALF, CHUNK * HALF)], msg[b], sem_l[b])

        def wait_loads(j, b):
            @pl.when(valid(j))
            def _():
                ck = base + j
                off = ck * CHUNK
                pltpu.make_async_copy(snd.at[pl.ds(off, CHUNK)], idx_s[b], sem_l[b]).wait()
                pltpu.make_async_copy(rcv.at[pl.ds(off, CHUNK)], idx_r[b], sem_l[b]).wait()
                pltpu.make_async_copy(ep_h.at[pl.ds(0, CHUNK * HALF)], msg[b], sem_l[b]).wait()

        def fire_gather(j, b):
            @pl.when(valid(j))
            def _():
                pltpu.async_copy(np_h.at[idx_s[b]], gat[b], sem_g[b])

        def process(j, b):
            @pl.when(valid(j))
            def _():
                pltpu.make_async_copy(np_h.at[idx_s[b]], gat[b], sem_g[b]).wait()

                def row_body(r, rc):
                    for e4 in range(PACK):
                        e = r * PACK + e4
                        for kk in range(HALF // 16):
                            src = pl.ds(r * 128 + e4 * HALF + kk * 16, 16)
                            dst = pl.ds(kk * 16, 16)
                            sct[b][e, dst] = jnp.maximum(
                                msg[b][src] + gat[b][e, dst], 0.0)
                    return rc

                lax.fori_loop(0, CROWS, row_body, 0)
                pltpu.sync_copy(sct[b], seg_sh.at[idx_r[b]], add=True)

        fire_loads(0, 0)
        fire_loads(1, 1)
        wait_loads(0, 0)
        fire_gather(0, 0)

        def loop(jo, carry):
            for b in range(2):
                j = jo * 2 + b
                wait_loads(j + 1, 1 - b)
                fire_gather(j + 1, 1 - b)
                process(j, b)
                fire_loads(j + 2, b)
            return carry

        lax.fori_loop(0, NJ2, loop, 0)
        plsc.subcore_barrier()
        pltpu.sync_copy(seg_sh.at[rows], seg_o.at[rows])

    def do_cnt_pass(cnt_o, ck_base):
        pltpu.sync_copy(zh.at[rows], seg_sh.at[rows])
        pltpu.sync_copy(ones_h, ones_v)
        plsc.subcore_barrier()
        cbase = ck_base + s * HJT

        def valid(j):
            return jnp.logical_and(j < HJT, cbase + j < ck_base + HCHUNKS)

        def fire(j, b):
            @pl.when(valid(j))
            def _():
                off = (cbase + j) * CHUNK
                pltpu.async_copy(rcv.at[pl.ds(off, CHUNK)], idx_r[b], sem_l[b])

        def process(j, b):
            @pl.when(valid(j))
            def _():
                off = (cbase + j) * CHUNK
                pltpu.make_async_copy(rcv.at[pl.ds(off, CHUNK)], idx_r[b], sem_l[b]).wait()
                pltpu.sync_copy(ones_v, seg_sh.at[idx_r[b]], add=True)

        fire(0, 0)
        fire(1, 1)

        def loop(jo, carry):
            for b in range(2):
                j = jo * 2 + b
                process(j, b)
                fire(j + 2, b)
            return carry

        lax.fori_loop(0, HJ2, loop, 0)
        plsc.subcore_barrier()
        pltpu.sync_copy(seg_sh.at[rows], cnt_o.at[rows])

    @pl.when(c == 0)
    def _():
        do_pass(ep0, np0, seg0_o)
        do_cnt_pass(cnta_o, 0)

    @pl.when(c == 1)
    def _():
        do_pass(ep1, np1, seg1_o)
        do_cnt_pass(cntb_o, HCHUNKS)


def _sc_scatter(eps, nps, senders, receivers, zh, ones32):
    mesh = plsc.VectorSubcoreMesh(
        core_axis_name="c", subcore_axis_name="s", num_cores=2, num_subcores=NS)
    return pl.kernel(
        _sc_body,
        out_type=tuple([jax.ShapeDtypeStruct((N_PAD, HALF), f32)] * 4),
        mesh=mesh,
        compiler_params=pltpu.CompilerParams(use_tc_tiling_on_sc=False),
        scratch_types=[
            pltpu.VMEM((CHUNK,), i32),
            pltpu.VMEM((CHUNK,), i32),
            pltpu.VMEM((CHUNK,), i32),
            pltpu.VMEM((CHUNK,), i32),
            pltpu.VMEM((CHUNK * HALF,), f32),
            pltpu.VMEM((CHUNK * HALF,), f32),
            pltpu.VMEM((CHUNK, HALF), f32),
            pltpu.VMEM((CHUNK, HALF), f32),
            pltpu.VMEM((CHUNK, HALF), f32),
            pltpu.VMEM((CHUNK, HALF), f32),
            pltpu.VMEM((CHUNK, HALF), f32),
            pltpu.VMEM_SHARED((N_PAD, HALF), f32),
            pltpu.SemaphoreType.DMA,
            pltpu.SemaphoreType.DMA,
            pltpu.SemaphoreType.DMA,
            pltpu.SemaphoreType.DMA,
        ],
    )(*eps, *nps, senders, receivers, zh, ones32)


def _update_body(seg0, seg1, cnta, cntb, n, bg, Wnn, Wni, Wng, bn_, Wge,
                 Wgn, Wgg, bg_, Wrn, brn, Wrg, brg, nout_o, gout_o, acc_e, acc_n):
    i = pl.program_id(0)
    nblocks = pl.num_programs(0)

    @pl.when(i == 0)
    def _():
        acc_e[...] = jnp.zeros_like(acc_e)
        acc_n[...] = jnp.zeros_like(acc_n)

    seg = jnp.concatenate([seg0[...], seg1[...]], axis=1)
    deg = jnp.maximum(cnta[:, 0:1] + cntb[:, 0:1], 1.0)
    inc = seg / deg
    u = jnp.maximum(bg[...], 0.0)
    cn = jnp.dot(u, Wng[...], preferred_element_type=f32) + bn_[...]
    n2 = jnp.maximum(
        jnp.dot(n[...], Wnn[...], preferred_element_type=f32)
        + jnp.dot(inc, Wni[...], preferred_element_type=f32) + cn, 0.0)
    nout_o[...] = _sigmoid(jnp.dot(n2, Wrn[...], preferred_element_type=f32) + brn[...])
    acc_e[...] += jnp.sum(seg, axis=0, keepdims=True)
    acc_n[...] += jnp.sum(n2, axis=0, keepdims=True)

    @pl.when(i == nblocks - 1)
    def _():
        mean_e2 = acc_e[...] * (1.0 / N_EDGES)
        mean_n2 = acc_n[...] * (1.0 / N_NODES)
        u2 = jnp.maximum(
            jnp.dot(mean_e2, Wge[...], preferred_element_type=f32)
            + jnp.dot(mean_n2, Wgn[...], preferred_element_type=f32)
            + jnp.dot(u, Wgg[...], preferred_element_type=f32) + bg_[...], 0.0)
        gout_o[...] = _sigmoid(jnp.dot(u2, Wrg[...], preferred_element_type=f32) + brg[...])


def _update(seg0, seg1, cnta, cntb, n, bg, Wn_n, Wn_i, Wn_g, bn_, Wg_e, Wg_n,
            Wg_g, bg_, Wr_n, br_n, Wr_g, br_g):
    grid = (N_NODES // NODE_BLK,)
    return pl.pallas_call(
        _update_body,
        grid=grid,
        in_specs=[pl.BlockSpec((NODE_BLK, HALF), lambda i: (i, 0))] * 4 + [
            pl.BlockSpec((NODE_BLK, 16), lambda i: (i, 0)),
            _full((1, 4)), _full((16, 128)), _full((64, 128)), _full((4, 128)),
            _full((1, 128)), _full((64, 32)), _full((128, 32)), _full((4, 32)),
            _full((1, 32)), _full((128, 1)), _full((1, 1)), _full((32, 1)),
            _full((1, 1)),
        ],
        out_specs=[
            pl.BlockSpec((NODE_BLK, 1), lambda i: (i, 0)),
            pl.BlockSpec((1, 1), lambda i: (0, 0)),
        ],
        out_shape=[
            jax.ShapeDtypeStruct((N_NODES, 1), f32),
            jax.ShapeDtypeStruct((1, 1), f32),
        ],
        scratch_shapes=[
            pltpu.VMEM((1, 64), f32),
            pltpu.VMEM((1, 128), f32),
        ],
    )(seg0, seg1, cnta, cntb, n, bg, Wn_n, Wn_i, Wn_g, bn_, Wg_e, Wg_n,
      Wg_g, bg_, Wr_n, br_n, Wr_g, br_g)


def kernel(x, edge_attr, senders, receivers, We1, be1, We2, be2, Wn1, bn1,
           Wn2, bn2, bg_enc, We_e, We_s, We_g, be, Wn_n, Wn_i, Wn_g, bn,
           Wg_e, Wg_n, Wg_g, bg, Wr_n, br_n, Wr_g, br_g):
    r = lambda v: v.reshape(1, -1)
    eps = _edge_encode(edge_attr, We1, be1, We2, be2, We_e, We_g, be, bg_enc)
    n_enc, np0, np1 = _node_encode(x, Wn1, r(bn1), Wn2, r(bn2), We_s)
    zh = jnp.zeros((N_PAD, HALF), f32)
    ones32 = jnp.ones((CHUNK, HALF), f32)
    eps_flat = tuple(e.reshape(-1) for e in eps)
    seg0, seg1, cnta, cntb = _sc_scatter(eps_flat, (np0, np1), senders,
                                         receivers, zh, ones32)
    node_out, global_out = _update(seg0, seg1, cnta, cntb, n_enc, r(bg_enc),
                                   Wn_n, Wn_i, Wn_g, r(bn), Wg_e, Wg_n, Wg_g,
                                   r(bg), Wr_n, r(br_n), Wr_g, r(br_g))
    return node_out, global_out


# speedup vs baseline: 3.9349x; 1.6151x over previous
import jax
import jax.numpy as jnp
from jax import lax
from jax.experimental import pallas as pl
from jax.experimental.pallas import tpu as pltpu
from jax.experimental.pallas import tpu_sc as plsc

N_NODES = 50000
N_EDGES = 800000
HALF = 32
PACK = 128 // HALF
EP_ROWS = N_EDGES // PACK
CHUNK = 128
CROWS = CHUNK // PACK
NCHUNKS = N_EDGES // CHUNK
HCHUNKS = NCHUNKS // 2
NS = 16
N_PAD = 50048
ROWS_PT = N_PAD // NS
EDGE_BLK = 6400
NODE_BLK = 1000
f32 = jnp.float32
i32 = jnp.int32


def _sigmoid(v):
    return 1.0 / (1.0 + jnp.exp(-v))


def _full(shape):
    n = len(shape)
    return pl.BlockSpec(shape, lambda i, _n=n: (0,) * _n)


def _edge_enc_body(ea0, ea1, W1b0, W1b1, b1b, W2b, b2b, Web0, Web1, Wgb0,
                   Wgb1, beb0, beb1, bg4, ep0_o, ep1_o):
    u4 = jnp.maximum(bg4[...], 0.0)
    h = jnp.maximum(jnp.dot(ea0[...], W1b0[...], preferred_element_type=f32)
                    + jnp.dot(ea1[...], W1b1[...], preferred_element_type=f32)
                    + b1b[...], 0.0)
    h = jnp.maximum(jnp.dot(h, W2b[...], preferred_element_type=f32) + b2b[...], 0.0)
    ce0 = jnp.dot(u4, Wgb0[...], preferred_element_type=f32) + beb0[...]
    ce1 = jnp.dot(u4, Wgb1[...], preferred_element_type=f32) + beb1[...]
    ep0_o[...] = jnp.dot(h, Web0[...], preferred_element_type=f32) + ce0
    ep1_o[...] = jnp.dot(h, Web1[...], preferred_element_type=f32) + ce1


def _bd4(W):
    r, c = W.shape
    out = jnp.zeros((4 * r, 4 * c), W.dtype)
    for k in range(4):
        out = out.at[k * r:(k + 1) * r, k * c:(k + 1) * c].set(W)
    return out


def _edge_encode(edge_attr, We1, be1, We2, be2, We_e, We_g, be, bg_enc):
    ea0 = edge_attr[:, 0].reshape(EP_ROWS, PACK)
    ea1 = edge_attr[:, 1].reshape(EP_ROWS, PACK)
    t4 = lambda v: jnp.tile(v.reshape(1, -1), (1, 4))
    grid = (EP_ROWS // (EDGE_BLK // PACK),)
    blk = EDGE_BLK // PACK
    return pl.pallas_call(
        _edge_enc_body,
        grid=grid,
        in_specs=[
            pl.BlockSpec((blk, PACK), lambda i: (i, 0)),
            pl.BlockSpec((blk, PACK), lambda i: (i, 0)),
            _full((PACK, 16)), _full((PACK, 16)), _full((1, 16)),
            _full((16, 32)), _full((1, 32)), _full((32, 128)),
            _full((32, 128)), _full((16, 128)), _full((16, 128)),
            _full((1, 128)), _full((1, 128)), _full((1, 16)),
        ],
        out_specs=[pl.BlockSpec((blk, 128), lambda i: (i, 0))] * 2,
        out_shape=[jax.ShapeDtypeStruct((EP_ROWS, 128), f32)] * 2,
    )(ea0, ea1, _bd4(We1[0:1, :]), _bd4(We1[1:2, :]), t4(be1), _bd4(We2),
      t4(be2), _bd4(We_e[:, :HALF]), _bd4(We_e[:, HALF:]),
      _bd4(We_g[:, :HALF]), _bd4(We_g[:, HALF:]), t4(be[:HALF]),
      t4(be[HALF:]), t4(bg_enc))


def _node_enc_body(x, Wn1, bn1, Wn2, bn2, Wes, n_o, np0_o, np1_o):
    h = jnp.maximum(jnp.dot(x[...], Wn1[...], preferred_element_type=f32) + bn1[...], 0.0)
    n = jnp.maximum(jnp.dot(h, Wn2[...], preferred_element_type=f32) + bn2[...], 0.0)
    npj = jnp.dot(n, Wes[...], preferred_element_type=f32)
    n_o[...] = n
    np0_o[...] = npj[:, :HALF]
    np1_o[...] = npj[:, HALF:]


def _node_encode(x, Wn1, bn1, Wn2, bn2, We_s):
    grid = (N_NODES // NODE_BLK,)
    return pl.pallas_call(
        _node_enc_body,
        grid=grid,
        in_specs=[
            pl.BlockSpec((NODE_BLK, 83), lambda i: (i, 0)),
            _full((83, 8)), _full((1, 8)), _full((8, 16)), _full((1, 16)),
            _full((16, 64)),
        ],
        out_specs=[pl.BlockSpec((NODE_BLK, 16), lambda i: (i, 0))]
        + [pl.BlockSpec((NODE_BLK, HALF), lambda i: (i, 0))] * 2,
        out_shape=[jax.ShapeDtypeStruct((N_NODES, 16), f32)]
        + [jax.ShapeDtypeStruct((N_PAD, HALF), f32)] * 2,
    )(x, Wn1, bn1, Wn2, bn2, We_s)


NJT = (NCHUNKS + NS - 1) // NS
NJ2 = (NJT + 1) // 2
HJT = (HCHUNKS + NS - 1) // NS
HJ2 = (HJT + 1) // 2


def _sc_body(ep0, ep1, np0, np1, snd, rcv, zh, ones_h,
             seg0_o, seg1_o, cnta_o, cntb_o,
             idx_s0, idx_s1, idx_r0, idx_r1, msg0, msg1, gat0, gat1,
             sct0, sct1, ones_v, seg_sh, sem_l0, sem_l1, sem_g0, sem_g1):
    s = lax.axis_index("s")
    c = lax.axis_index("c")
    r0 = s * ROWS_PT
    rows = pl.ds(r0, ROWS_PT)
    base = s * NJT
    idx_s = (idx_s0, idx_s1)
    idx_r = (idx_r0, idx_r1)
    msg = (msg0, msg1)
    gat = (gat0, gat1)
    sct = (sct0, sct1)
    sem_l = (sem_l0, sem_l1)
    sem_g = (sem_g0, sem_g1)

    def do_pass(ep_h, np_h, seg_o):
        pltpu.sync_copy(zh.at[rows], seg_sh.at[rows])
        plsc.subcore_barrier()

        def valid(j):
            return jnp.logical_and(j < NJT, base + j < NCHUNKS)

        def fire_loads(j, b):
            @pl.when(valid(j))
            def _():
                ck = base + j
                off = ck * CHUNK
                pltpu.async_copy(snd.at[pl.ds(off, CHUNK)], idx_s[b], sem_l[b])
                pltpu.async_copy(rcv.at[pl.ds(off, CHUNK)], idx_r[b], sem_l[b])
                pltpu.async_copy(ep_h.at[pl.ds(ck * CHUNK * HALF, CHUNK * HALF)], msg[b], sem_l[b])

        def wait_loads(j, b):
            @pl.when(valid(j))
            def _():
                ck = base + j
                off = ck * CHUNK
                pltpu.make_async_copy(snd.at[pl.ds(off, CHUNK)], idx_s[b], sem_l[b]).wait()
                pltpu.make_async_copy(rcv.at[pl.ds(off, CHUNK)], idx_r[b], sem_l[b]).wait()
                pltpu.make_async_copy(ep_h.at[pl.ds(0, CHUNK * HALF)], msg[b], sem_l[b]).wait()

        def fire_gather(j, b):
            @pl.when(valid(j))
            def _():
                pltpu.async_copy(np_h.at[idx_s[b]], gat[b], sem_g[b])

        def process(j, b):
            @pl.when(valid(j))
            def _():
                pltpu.make_async_copy(np_h.at[idx_s[b]], gat[b], sem_g[b]).wait()

                def row_body(r, rc):
                    for e4 in range(PACK):
                        e = r * PACK + e4
                        for kk in range(HALF // 16):
                            src = pl.ds(r * 128 + e4 * HALF + kk * 16, 16)
                            dst = pl.ds(kk * 16, 16)
                            sct[b][e, dst] = jnp.maximum(
                                msg[b][src] + gat[b][e, dst], 0.0)
                    return rc

                lax.fori_loop(0, CROWS, row_body, 0)
                pltpu.sync_copy(sct[b], seg_sh.at[idx_r[b]], add=True)

        fire_loads(0, 0)
        fire_loads(1, 1)
        wait_loads(0, 0)
        fire_gather(0, 0)

        def loop(jo, carry):
            for b in range(2):
                j = jo * 2 + b
                wait_loads(j + 1, 1 - b)
                fire_gather(j + 1, 1 - b)
                process(j, b)
                fire_loads(j + 2, b)
            return carry

        lax.fori_loop(0, NJ2, loop, 0)
        plsc.subcore_barrier()
        pltpu.sync_copy(seg_sh.at[rows], seg_o.at[rows])

    def do_cnt_pass(cnt_o, ck_base):
        pltpu.sync_copy(zh.at[rows], seg_sh.at[rows])
        pltpu.sync_copy(ones_h, ones_v)
        plsc.subcore_barrier()
        cbase = ck_base + s * HJT

        def valid(j):
            return jnp.logical_and(j < HJT, cbase + j < ck_base + HCHUNKS)

        def fire(j, b):
            @pl.when(valid(j))
            def _():
                off = (cbase + j) * CHUNK
                pltpu.async_copy(rcv.at[pl.ds(off, CHUNK)], idx_r[b], sem_l[b])

        def process(j, b):
            @pl.when(valid(j))
            def _():
                off = (cbase + j) * CHUNK
                pltpu.make_async_copy(rcv.at[pl.ds(off, CHUNK)], idx_r[b], sem_l[b]).wait()
                pltpu.sync_copy(ones_v, seg_sh.at[idx_r[b]], add=True)

        fire(0, 0)
        fire(1, 1)

        def loop(jo, carry):
            for b in range(2):
                j = jo * 2 + b
                process(j, b)
                fire(j + 2, b)
            return carry

        lax.fori_loop(0, HJ2, loop, 0)
        plsc.subcore_barrier()
        pltpu.sync_copy(seg_sh.at[rows], cnt_o.at[rows])

    @pl.when(c == 0)
    def _():
        do_pass(ep0, np0, seg0_o)
        do_cnt_pass(cnta_o, 0)

    @pl.when(c == 1)
    def _():
        do_pass(ep1, np1, seg1_o)
        do_cnt_pass(cntb_o, HCHUNKS)


def _sc_scatter(eps, nps, senders, receivers, zh, ones32):
    mesh = plsc.VectorSubcoreMesh(
        core_axis_name="c", subcore_axis_name="s", num_cores=2, num_subcores=NS)
    return pl.kernel(
        _sc_body,
        out_type=tuple([jax.ShapeDtypeStruct((N_PAD, HALF), f32)] * 4),
        mesh=mesh,
        compiler_params=pltpu.CompilerParams(use_tc_tiling_on_sc=False),
        scratch_types=[
            pltpu.VMEM((CHUNK,), i32),
            pltpu.VMEM((CHUNK,), i32),
            pltpu.VMEM((CHUNK,), i32),
            pltpu.VMEM((CHUNK,), i32),
            pltpu.VMEM((CHUNK * HALF,), f32),
            pltpu.VMEM((CHUNK * HALF,), f32),
            pltpu.VMEM((CHUNK, HALF), f32),
            pltpu.VMEM((CHUNK, HALF), f32),
            pltpu.VMEM((CHUNK, HALF), f32),
            pltpu.VMEM((CHUNK, HALF), f32),
            pltpu.VMEM((CHUNK, HALF), f32),
            pltpu.VMEM_SHARED((N_PAD, HALF), f32),
            pltpu.SemaphoreType.DMA,
            pltpu.SemaphoreType.DMA,
            pltpu.SemaphoreType.DMA,
            pltpu.SemaphoreType.DMA,
        ],
    )(*eps, *nps, senders, receivers, zh, ones32)


def _update_body(seg0, seg1, cnta, cntb, n, bg, Wnn, Wni, Wng, bn_, Wge,
                 Wgn, Wgg, bg_, Wrn, brn, Wrg, brg, nout_o, gout_o, acc_e, acc_n):
    i = pl.program_id(0)
    nblocks = pl.num_programs(0)

    @pl.when(i == 0)
    def _():
        acc_e[...] = jnp.zeros_like(acc_e)
        acc_n[...] = jnp.zeros_like(acc_n)

    seg = jnp.concatenate([seg0[...], seg1[...]], axis=1)
    deg = jnp.maximum(cnta[:, 0:1] + cntb[:, 0:1], 1.0)
    inc = seg / deg
    u = jnp.maximum(bg[...], 0.0)
    cn = jnp.dot(u, Wng[...], preferred_element_type=f32) + bn_[...]
    n2 = jnp.maximum(
        jnp.dot(n[...], Wnn[...], preferred_element_type=f32)
        + jnp.dot(inc, Wni[...], preferred_element_type=f32) + cn, 0.0)
    nout_o[...] = _sigmoid(jnp.dot(n2, Wrn[...], preferred_element_type=f32) + brn[...])
    acc_e[...] += jnp.sum(seg, axis=0, keepdims=True)
    acc_n[...] += jnp.sum(n2, axis=0, keepdims=True)

    @pl.when(i == nblocks - 1)
    def _():
        mean_e2 = acc_e[...] * (1.0 / N_EDGES)
        mean_n2 = acc_n[...] * (1.0 / N_NODES)
        u2 = jnp.maximum(
            jnp.dot(mean_e2, Wge[...], preferred_element_type=f32)
            + jnp.dot(mean_n2, Wgn[...], preferred_element_type=f32)
            + jnp.dot(u, Wgg[...], preferred_element_type=f32) + bg_[...], 0.0)
        gout_o[...] = _sigmoid(jnp.dot(u2, Wrg[...], preferred_element_type=f32) + brg[...])


def _update(seg0, seg1, cnta, cntb, n, bg, Wn_n, Wn_i, Wn_g, bn_, Wg_e, Wg_n,
            Wg_g, bg_, Wr_n, br_n, Wr_g, br_g):
    grid = (N_NODES // NODE_BLK,)
    return pl.pallas_call(
        _update_body,
        grid=grid,
        in_specs=[pl.BlockSpec((NODE_BLK, HALF), lambda i: (i, 0))] * 4 + [
            pl.BlockSpec((NODE_BLK, 16), lambda i: (i, 0)),
            _full((1, 4)), _full((16, 128)), _full((64, 128)), _full((4, 128)),
            _full((1, 128)), _full((64, 32)), _full((128, 32)), _full((4, 32)),
            _full((1, 32)), _full((128, 1)), _full((1, 1)), _full((32, 1)),
            _full((1, 1)),
        ],
        out_specs=[
            pl.BlockSpec((NODE_BLK, 1), lambda i: (i, 0)),
            pl.BlockSpec((1, 1), lambda i: (0, 0)),
        ],
        out_shape=[
            jax.ShapeDtypeStruct((N_NODES, 1), f32),
            jax.ShapeDtypeStruct((1, 1), f32),
        ],
        scratch_shapes=[
            pltpu.VMEM((1, 64), f32),
            pltpu.VMEM((1, 128), f32),
        ],
    )(seg0, seg1, cnta, cntb, n, bg, Wn_n, Wn_i, Wn_g, bn_, Wg_e, Wg_n,
      Wg_g, bg_, Wr_n, br_n, Wr_g, br_g)


def kernel(x, edge_attr, senders, receivers, We1, be1, We2, be2, Wn1, bn1,
           Wn2, bn2, bg_enc, We_e, We_s, We_g, be, Wn_n, Wn_i, Wn_g, bn,
           Wg_e, Wg_n, Wg_g, bg, Wr_n, br_n, Wr_g, br_g):
    r = lambda v: v.reshape(1, -1)
    eps = _edge_encode(edge_attr, We1, be1, We2, be2, We_e, We_g, be, bg_enc)
    n_enc, np0, np1 = _node_encode(x, Wn1, r(bn1), Wn2, r(bn2), We_s)
    zh = jnp.zeros((N_PAD, HALF), f32)
    ones32 = jnp.ones((CHUNK, HALF), f32)
    eps_flat = tuple(e.reshape(-1) for e in eps)
    seg0, seg1, cnta, cntb = _sc_scatter(eps_flat, (np0, np1), senders,
                                         receivers, zh, ones32)
    node_out, global_out = _update(seg0, seg1, cnta, cntb, n_enc, r(bg_enc),
                                   Wn_n, Wn_i, Wn_g, r(bn), Wg_e, Wg_n, Wg_g,
                                   r(bg), Wr_n, r(br_n), Wr_g, r(br_g))
    return node_out, global_out


# speedup vs baseline: 4.1614x; 1.0576x over previous
import jax
import jax.numpy as jnp
from jax import lax
from jax.experimental import pallas as pl
from jax.experimental.pallas import tpu as pltpu
from jax.experimental.pallas import tpu_sc as plsc

N_NODES = 50000
N_EDGES = 800000
HALF = 32
PACK = 128 // HALF
EP_ROWS = N_EDGES // PACK
CHUNK = 128
CROWS = CHUNK // PACK
NCHUNKS = N_EDGES // CHUNK
HCHUNKS = NCHUNKS // 2
NS = 16
N_PAD = 50048
ROWS_PT = N_PAD // NS
EDGE_BLK = 6400
NODE_BLK = 1000
f32 = jnp.float32
i32 = jnp.int32


def _sigmoid(v):
    return 1.0 / (1.0 + jnp.exp(-v))


def _full(shape):
    n = len(shape)
    return pl.BlockSpec(shape, lambda i, _n=n: (0,) * _n)


def _edge_enc_body(ea0, ea1, W1b0, W1b1, b1b, W2b, b2b, Web0, Web1, Wgb0,
                   Wgb1, beb0, beb1, bg4, ep0_o, ep1_o):
    u4 = jnp.maximum(bg4[...], 0.0)
    h = jnp.maximum(jnp.dot(ea0[...], W1b0[...], preferred_element_type=f32)
                    + jnp.dot(ea1[...], W1b1[...], preferred_element_type=f32)
                    + b1b[...], 0.0)
    h = jnp.maximum(jnp.dot(h, W2b[...], preferred_element_type=f32) + b2b[...], 0.0)
    ce0 = jnp.dot(u4, Wgb0[...], preferred_element_type=f32) + beb0[...]
    ce1 = jnp.dot(u4, Wgb1[...], preferred_element_type=f32) + beb1[...]
    ep0_o[...] = jnp.dot(h, Web0[...], preferred_element_type=f32) + ce0
    ep1_o[...] = jnp.dot(h, Web1[...], preferred_element_type=f32) + ce1


def _bd4(W):
    r, c = W.shape
    out = jnp.zeros((4 * r, 4 * c), W.dtype)
    for k in range(4):
        out = out.at[k * r:(k + 1) * r, k * c:(k + 1) * c].set(W)
    return out


def _edge_encode(edge_attr, We1, be1, We2, be2, We_e, We_g, be, bg_enc):
    ea0 = edge_attr[:, 0].reshape(EP_ROWS, PACK)
    ea1 = edge_attr[:, 1].reshape(EP_ROWS, PACK)
    t4 = lambda v: jnp.tile(v.reshape(1, -1), (1, 4))
    grid = (EP_ROWS // (EDGE_BLK // PACK),)
    blk = EDGE_BLK // PACK
    return pl.pallas_call(
        _edge_enc_body,
        grid=grid,
        in_specs=[
            pl.BlockSpec((blk, PACK), lambda i: (i, 0)),
            pl.BlockSpec((blk, PACK), lambda i: (i, 0)),
            _full((PACK, 16)), _full((PACK, 16)), _full((1, 16)),
            _full((16, 32)), _full((1, 32)), _full((32, 128)),
            _full((32, 128)), _full((16, 128)), _full((16, 128)),
            _full((1, 128)), _full((1, 128)), _full((1, 16)),
        ],
        out_specs=[pl.BlockSpec((blk, 128), lambda i: (i, 0))] * 2,
        out_shape=[jax.ShapeDtypeStruct((EP_ROWS, 128), f32)] * 2,
    )(ea0, ea1, _bd4(We1[0:1, :]), _bd4(We1[1:2, :]), t4(be1), _bd4(We2),
      t4(be2), _bd4(We_e[:, :HALF]), _bd4(We_e[:, HALF:]),
      _bd4(We_g[:, :HALF]), _bd4(We_g[:, HALF:]), t4(be[:HALF]),
      t4(be[HALF:]), t4(bg_enc))


def _node_enc_body(x, Wn1, bn1, Wn2, bn2, Wes, n_o, np0_o, np1_o):
    h = jnp.maximum(jnp.dot(x[...], Wn1[...], preferred_element_type=f32) + bn1[...], 0.0)
    n = jnp.maximum(jnp.dot(h, Wn2[...], preferred_element_type=f32) + bn2[...], 0.0)
    npj = jnp.dot(n, Wes[...], preferred_element_type=f32)
    n_o[...] = n
    np0_o[...] = npj[:, :HALF]
    np1_o[...] = npj[:, HALF:]


def _node_encode(x, Wn1, bn1, Wn2, bn2, We_s):
    grid = (N_NODES // NODE_BLK,)
    return pl.pallas_call(
        _node_enc_body,
        grid=grid,
        in_specs=[
            pl.BlockSpec((NODE_BLK, 83), lambda i: (i, 0)),
            _full((83, 8)), _full((1, 8)), _full((8, 16)), _full((1, 16)),
            _full((16, 64)),
        ],
        out_specs=[pl.BlockSpec((NODE_BLK, 16), lambda i: (i, 0))]
        + [pl.BlockSpec((NODE_BLK, HALF), lambda i: (i, 0))] * 2,
        out_shape=[jax.ShapeDtypeStruct((N_NODES, 16), f32)]
        + [jax.ShapeDtypeStruct((N_PAD, HALF), f32)] * 2,
    )(x, Wn1, bn1, Wn2, bn2, We_s)


NJT = (NCHUNKS + NS - 1) // NS


def _sc_body(ep0, ep1, np0, np1, snd, rcv, zh, ones_h,
             seg0_o, seg1_o, cnta_o, cntb_o,
             idx_s0, idx_s1, idx_r0, idx_r1, idx_r2, idx_r3,
             msg0, msg1, gat0, gat1, sct0, sct1, ones_v, seg_sh,
             sem_l0, sem_l1, sem_g0, sem_g1, sem_s0, sem_s1):
    s = lax.axis_index("s")
    c = lax.axis_index("c")
    r0 = s * ROWS_PT
    rows = pl.ds(r0, ROWS_PT)
    base = s * NJT
    idx_s = (idx_s0, idx_s1)
    idx_r = (idx_r0, idx_r1, idx_r2, idx_r3)
    msg = (msg0, msg1)
    gat = (gat0, gat1)
    sct = (sct0, sct1)
    sem_l = (sem_l0, sem_l1)
    sem_g = (sem_g0, sem_g1)
    sem_s = (sem_s0, sem_s1)

    def do_pass(ep_h, np_h, seg_o):
        pltpu.sync_copy(zh.at[rows], seg_sh.at[rows])
        plsc.subcore_barrier()

        def valid(j):
            return jnp.logical_and(jnp.logical_and(j >= 0, j < NJT),
                                   base + j < NCHUNKS)

        def fire_loads(j, ls, rs):
            @pl.when(valid(j))
            def _():
                ck = base + j
                off = ck * CHUNK
                pltpu.async_copy(snd.at[pl.ds(off, CHUNK)], idx_s[ls], sem_l[ls])
                pltpu.async_copy(rcv.at[pl.ds(off, CHUNK)], idx_r[rs], sem_l[ls])
                pltpu.async_copy(ep_h.at[pl.ds(ck * CHUNK * HALF, CHUNK * HALF)],
                                 msg[ls], sem_l[ls])

        def wait_loads(j, ls, rs):
            @pl.when(valid(j))
            def _():
                ck = base + j
                off = ck * CHUNK
                pltpu.make_async_copy(snd.at[pl.ds(off, CHUNK)], idx_s[ls], sem_l[ls]).wait()
                pltpu.make_async_copy(rcv.at[pl.ds(off, CHUNK)], idx_r[rs], sem_l[ls]).wait()
                pltpu.make_async_copy(ep_h.at[pl.ds(0, CHUNK * HALF)], msg[ls], sem_l[ls]).wait()

        def fire_gather(j, ls):
            @pl.when(valid(j))
            def _():
                pltpu.async_copy(np_h.at[idx_s[ls]], gat[ls], sem_g[ls])

        def wait_scatter(j, ls, rs):
            @pl.when(valid(j))
            def _():
                pltpu.make_async_copy(sct[ls], seg_sh.at[idx_r[rs]], sem_s[ls]).wait()

        def process(j, ls, rs):
            wait_scatter(j - 2, ls, (rs + 2) % 4)

            @pl.when(valid(j))
            def _():
                pltpu.make_async_copy(np_h.at[idx_s[ls]], gat[ls], sem_g[ls]).wait()

                def row_body(r, rc):
                    for e4 in range(PACK):
                        e = r * PACK + e4
                        for kk in range(HALF // 16):
                            src = pl.ds(r * 128 + e4 * HALF + kk * 16, 16)
                            dst = pl.ds(kk * 16, 16)
                            sct[ls][e, dst] = jnp.maximum(
                                msg[ls][src] + gat[ls][e, dst], 0.0)
                    return rc

                lax.fori_loop(0, CROWS, row_body, 0)
                pltpu.async_copy(sct[ls], seg_sh.at[idx_r[rs]], sem_s[ls], add=True)

        fire_loads(0, 0, 0)
        fire_loads(1, 1, 1)
        wait_loads(0, 0, 0)
        fire_gather(0, 0)

        def loop(jo, carry):
            for b in range(4):
                j = jo * 4 + b
                wait_loads(j + 1, (b + 1) % 2, (b + 1) % 4)
                fire_gather(j + 1, (b + 1) % 2)
                process(j, b % 2, b)
                fire_loads(j + 2, b % 2, (b + 2) % 4)
            return carry

        NJ4 = (NJT + 3) // 4
        lax.fori_loop(0, NJ4, loop, 0)
        wait_scatter(4 * NJ4 - 2, 0, 2)
        wait_scatter(4 * NJ4 - 1, 1, 3)
        plsc.subcore_barrier()
        pltpu.sync_copy(seg_sh.at[rows], seg_o.at[rows])

    def do_cnt_pass(cnt_o, ck_base):
        pltpu.sync_copy(zh.at[rows], seg_sh.at[rows])
        pltpu.sync_copy(ones_h, ones_v)
        plsc.subcore_barrier()
        HJT = (HCHUNKS + NS - 1) // NS
        cbase = ck_base + s * HJT

        def valid(j):
            return jnp.logical_and(j < HJT, cbase + j < ck_base + HCHUNKS)

        def fire(j, b):
            @pl.when(valid(j))
            def _():
                off = (cbase + j) * CHUNK
                pltpu.async_copy(rcv.at[pl.ds(off, CHUNK)], idx_r[b], sem_l[b])

        def process(j, b):
            @pl.when(valid(j))
            def _():
                off = (cbase + j) * CHUNK
                pltpu.make_async_copy(rcv.at[pl.ds(off, CHUNK)], idx_r[b], sem_l[b]).wait()
                pltpu.sync_copy(ones_v, seg_sh.at[idx_r[b]], add=True)

        fire(0, 0)
        fire(1, 1)

        def loop(jo, carry):
            for b in range(2):
                j = jo * 2 + b
                process(j, b)
                fire(j + 2, b)
            return carry

        lax.fori_loop(0, (HJT + 1) // 2, loop, 0)
        plsc.subcore_barrier()
        pltpu.sync_copy(seg_sh.at[rows], cnt_o.at[rows])

    @pl.when(c == 0)
    def _():
        do_pass(ep0, np0, seg0_o)
        do_cnt_pass(cnta_o, 0)

    @pl.when(c == 1)
    def _():
        do_pass(ep1, np1, seg1_o)
        do_cnt_pass(cntb_o, HCHUNKS)


def _sc_scatter(eps, nps, senders, receivers, zh, ones8):
    mesh = plsc.VectorSubcoreMesh(
        core_axis_name="c", subcore_axis_name="s", num_cores=2, num_subcores=NS)
    return pl.kernel(
        _sc_body,
        out_type=tuple([jax.ShapeDtypeStruct((N_PAD, HALF), f32)] * 4),
        mesh=mesh,
        compiler_params=pltpu.CompilerParams(use_tc_tiling_on_sc=False),
        scratch_types=[
            pltpu.VMEM((CHUNK,), i32),
            pltpu.VMEM((CHUNK,), i32),
            pltpu.VMEM((CHUNK,), i32),
            pltpu.VMEM((CHUNK,), i32),
            pltpu.VMEM((CHUNK,), i32),
            pltpu.VMEM((CHUNK,), i32),
            pltpu.VMEM((CHUNK * HALF,), f32),
            pltpu.VMEM((CHUNK * HALF,), f32),
            pltpu.VMEM((CHUNK, HALF), f32),
            pltpu.VMEM((CHUNK, HALF), f32),
            pltpu.VMEM((CHUNK, HALF), f32),
            pltpu.VMEM((CHUNK, HALF), f32),
            pltpu.VMEM((CHUNK, HALF), f32),
            pltpu.VMEM_SHARED((N_PAD, HALF), f32),
            pltpu.SemaphoreType.DMA,
            pltpu.SemaphoreType.DMA,
            pltpu.SemaphoreType.DMA,
            pltpu.SemaphoreType.DMA,
            pltpu.SemaphoreType.DMA,
            pltpu.SemaphoreType.DMA,
        ],
    )(*eps, *nps, senders, receivers, zh, ones8)


def _update_body(seg0, seg1, cnta, cntb, n, bg, Wnn, Wni, Wng, bn_, Wge,
                 Wgn, Wgg, bg_, Wrn, brn, Wrg, brg, nout_o, gout_o, acc_e, acc_n):
    i = pl.program_id(0)
    nblocks = pl.num_programs(0)

    @pl.when(i == 0)
    def _():
        acc_e[...] = jnp.zeros_like(acc_e)
        acc_n[...] = jnp.zeros_like(acc_n)

    seg = jnp.concatenate([seg0[...], seg1[...]], axis=1)
    deg = jnp.maximum(cnta[:, 0:1] + cntb[:, 0:1], 1.0)
    inc = seg / deg
    u = jnp.maximum(bg[...], 0.0)
    cn = jnp.dot(u, Wng[...], preferred_element_type=f32) + bn_[...]
    n2 = jnp.maximum(
        jnp.dot(n[...], Wnn[...], preferred_element_type=f32)
        + jnp.dot(inc, Wni[...], preferred_element_type=f32) + cn, 0.0)
    nout_o[...] = _sigmoid(jnp.dot(n2, Wrn[...], preferred_element_type=f32) + brn[...])
    acc_e[...] += jnp.sum(seg, axis=0, keepdims=True)
    acc_n[...] += jnp.sum(n2, axis=0, keepdims=True)

    @pl.when(i == nblocks - 1)
    def _():
        mean_e2 = acc_e[...] * (1.0 / N_EDGES)
        mean_n2 = acc_n[...] * (1.0 / N_NODES)
        u2 = jnp.maximum(
            jnp.dot(mean_e2, Wge[...], preferred_element_type=f32)
            + jnp.dot(mean_n2, Wgn[...], preferred_element_type=f32)
            + jnp.dot(u, Wgg[...], preferred_element_type=f32) + bg_[...], 0.0)
        gout_o[...] = _sigmoid(jnp.dot(u2, Wrg[...], preferred_element_type=f32) + brg[...])


def _update(seg0, seg1, cnta, cntb, n, bg, Wn_n, Wn_i, Wn_g, bn_, Wg_e, Wg_n,
            Wg_g, bg_, Wr_n, br_n, Wr_g, br_g):
    grid = (N_NODES // NODE_BLK,)
    return pl.pallas_call(
        _update_body,
        grid=grid,
        in_specs=[pl.BlockSpec((NODE_BLK, HALF), lambda i: (i, 0))] * 4 + [
            pl.BlockSpec((NODE_BLK, 16), lambda i: (i, 0)),
            _full((1, 4)), _full((16, 128)), _full((64, 128)), _full((4, 128)),
            _full((1, 128)), _full((64, 32)), _full((128, 32)), _full((4, 32)),
            _full((1, 32)), _full((128, 1)), _full((1, 1)), _full((32, 1)),
            _full((1, 1)),
        ],
        out_specs=[
            pl.BlockSpec((NODE_BLK, 1), lambda i: (i, 0)),
            pl.BlockSpec((1, 1), lambda i: (0, 0)),
        ],
        out_shape=[
            jax.ShapeDtypeStruct((N_NODES, 1), f32),
            jax.ShapeDtypeStruct((1, 1), f32),
        ],
        scratch_shapes=[
            pltpu.VMEM((1, 64), f32),
            pltpu.VMEM((1, 128), f32),
        ],
    )(seg0, seg1, cnta, cntb, n, bg, Wn_n, Wn_i, Wn_g, bn_, Wg_e, Wg_n,
      Wg_g, bg_, Wr_n, br_n, Wr_g, br_g)


def kernel(x, edge_attr, senders, receivers, We1, be1, We2, be2, Wn1, bn1,
           Wn2, bn2, bg_enc, We_e, We_s, We_g, be, Wn_n, Wn_i, Wn_g, bn,
           Wg_e, Wg_n, Wg_g, bg, Wr_n, br_n, Wr_g, br_g):
    r = lambda v: v.reshape(1, -1)
    eps = _edge_encode(edge_attr, We1, be1, We2, be2, We_e, We_g, be, bg_enc)
    n_enc, np0, np1 = _node_encode(x, Wn1, r(bn1), Wn2, r(bn2), We_s)
    zh = jnp.zeros((N_PAD, HALF), f32)
    ones32 = jnp.ones((CHUNK, HALF), f32)
    eps_flat = tuple(e.reshape(-1) for e in eps)
    seg0, seg1, cnta, cntb = _sc_scatter(eps_flat, (np0, np1), senders,
                                         receivers, zh, ones32)
    node_out, global_out = _update(seg0, seg1, cnta, cntb, n_enc, r(bg_enc),
                                   Wn_n, Wn_i, Wn_g, r(bn), Wg_e, Wg_n, Wg_g,
                                   r(bg), Wr_n, r(br_n), Wr_g, r(br_g))
    return node_out, global_out


# speedup vs baseline: 4.1624x; 1.0002x over previous
import jax
import jax.numpy as jnp
from jax import lax
from jax.experimental import pallas as pl
from jax.experimental.pallas import tpu as pltpu
from jax.experimental.pallas import tpu_sc as plsc

N_NODES = 50000
N_EDGES = 800000
HALF = 32
PACK = 128 // HALF
EP_ROWS = N_EDGES // PACK
CHUNK = 128
CROWS = CHUNK // PACK
NCHUNKS = N_EDGES // CHUNK
HCHUNKS = NCHUNKS // 2
NS = 16
N_PAD = 50048
ROWS_PT = N_PAD // NS
EDGE_BLK = 6400
NODE_BLK = 2944
f32 = jnp.float32
i32 = jnp.int32


def _sigmoid(v):
    return 1.0 / (1.0 + jnp.exp(-v))


def _full(shape):
    n = len(shape)
    return pl.BlockSpec(shape, lambda i, _n=n: (0,) * _n)


def _edge_enc_body(ea0, ea1, W1b0, W1b1, b1b, W2b, b2b, Web0, Web1, Wgb0,
                   Wgb1, beb0, beb1, bg4, ep0_o, ep1_o):
    u4 = jnp.maximum(bg4[...], 0.0)
    h = jnp.maximum(jnp.dot(ea0[...], W1b0[...], preferred_element_type=f32)
                    + jnp.dot(ea1[...], W1b1[...], preferred_element_type=f32)
                    + b1b[...], 0.0)
    h = jnp.maximum(jnp.dot(h, W2b[...], preferred_element_type=f32) + b2b[...], 0.0)
    ce0 = jnp.dot(u4, Wgb0[...], preferred_element_type=f32) + beb0[...]
    ce1 = jnp.dot(u4, Wgb1[...], preferred_element_type=f32) + beb1[...]
    ep0_o[...] = jnp.dot(h, Web0[...], preferred_element_type=f32) + ce0
    ep1_o[...] = jnp.dot(h, Web1[...], preferred_element_type=f32) + ce1


def _bd4(W):
    r, c = W.shape
    out = jnp.zeros((4 * r, 4 * c), W.dtype)
    for k in range(4):
        out = out.at[k * r:(k + 1) * r, k * c:(k + 1) * c].set(W)
    return out


def _edge_encode(edge_attr, We1, be1, We2, be2, We_e, We_g, be, bg_enc):
    ea0 = edge_attr[:, 0].reshape(EP_ROWS, PACK)
    ea1 = edge_attr[:, 1].reshape(EP_ROWS, PACK)
    t4 = lambda v: jnp.tile(v.reshape(1, -1), (1, 4))
    grid = (EP_ROWS // (EDGE_BLK // PACK),)
    blk = EDGE_BLK // PACK
    return pl.pallas_call(
        _edge_enc_body,
        grid=grid,
        in_specs=[
            pl.BlockSpec((blk, PACK), lambda i: (i, 0)),
            pl.BlockSpec((blk, PACK), lambda i: (i, 0)),
            _full((PACK, 16)), _full((PACK, 16)), _full((1, 16)),
            _full((16, 32)), _full((1, 32)), _full((32, 128)),
            _full((32, 128)), _full((16, 128)), _full((16, 128)),
            _full((1, 128)), _full((1, 128)), _full((1, 16)),
        ],
        out_specs=[pl.BlockSpec((blk, 128), lambda i: (i, 0))] * 2,
        out_shape=[jax.ShapeDtypeStruct((EP_ROWS, 128), f32)] * 2,
    )(ea0, ea1, _bd4(We1[0:1, :]), _bd4(We1[1:2, :]), t4(be1), _bd4(We2),
      t4(be2), _bd4(We_e[:, :HALF]), _bd4(We_e[:, HALF:]),
      _bd4(We_g[:, :HALF]), _bd4(We_g[:, HALF:]), t4(be[:HALF]),
      t4(be[HALF:]), t4(bg_enc))


def _node_enc_body(x, Wn1, bn1, Wn2, bn2, Wes, n_o, np0_o, np1_o):
    h = jnp.maximum(jnp.dot(x[...], Wn1[...], preferred_element_type=f32) + bn1[...], 0.0)
    n = jnp.maximum(jnp.dot(h, Wn2[...], preferred_element_type=f32) + bn2[...], 0.0)
    npj = jnp.dot(n, Wes[...], preferred_element_type=f32)
    n_o[...] = n
    np0_o[...] = npj[:, :HALF]
    np1_o[...] = npj[:, HALF:]


NENC_BLK = 1000


def _node_encode(x, Wn1, bn1, Wn2, bn2, We_s):
    grid = (N_NODES // NENC_BLK,)
    return pl.pallas_call(
        _node_enc_body,
        grid=grid,
        in_specs=[
            pl.BlockSpec((NENC_BLK, 83), lambda i: (i, 0)),
            _full((83, 8)), _full((1, 8)), _full((8, 16)), _full((1, 16)),
            _full((16, 64)),
        ],
        out_specs=[pl.BlockSpec((NENC_BLK, 16), lambda i: (i, 0))]
        + [pl.BlockSpec((NENC_BLK, HALF), lambda i: (i, 0))] * 2,
        out_shape=[jax.ShapeDtypeStruct((N_NODES, 16), f32)]
        + [jax.ShapeDtypeStruct((N_PAD, HALF), f32)] * 2,
    )(x, Wn1, bn1, Wn2, bn2, We_s)


NJT = (NCHUNKS + NS - 1) // NS


def _sc_body(ep0, ep1, np0, np1, snd, rcv, zh, ones_h,
             seg0_o, seg1_o, cnta_o, cntb_o,
             idx_s0, idx_s1, idx_r0, idx_r1, idx_r2, idx_r3,
             msg0, msg1, gat0, gat1, sct0, sct1, ones_v, seg_sh,
             sem_l0, sem_l1, sem_g0, sem_g1, sem_s0, sem_s1):
    s = lax.axis_index("s")
    c = lax.axis_index("c")
    r0 = s * ROWS_PT
    rows = pl.ds(r0, ROWS_PT)
    base = s * NJT
    idx_s = (idx_s0, idx_s1)
    idx_r = (idx_r0, idx_r1, idx_r2, idx_r3)
    msg = (msg0, msg1)
    gat = (gat0, gat1)
    sct = (sct0, sct1)
    sem_l = (sem_l0, sem_l1)
    sem_g = (sem_g0, sem_g1)
    sem_s = (sem_s0, sem_s1)

    def do_pass(ep_h, np_h, seg_o):
        pltpu.sync_copy(zh.at[rows], seg_sh.at[rows])
        plsc.subcore_barrier()

        def valid(j):
            return jnp.logical_and(jnp.logical_and(j >= 0, j < NJT),
                                   base + j < NCHUNKS)

        def fire_loads(j, ls, rs):
            @pl.when(valid(j))
            def _():
                ck = base + j
                off = ck * CHUNK
                pltpu.async_copy(snd.at[pl.ds(off, CHUNK)], idx_s[ls], sem_l[ls])
                pltpu.async_copy(rcv.at[pl.ds(off, CHUNK)], idx_r[rs], sem_l[ls])
                pltpu.async_copy(ep_h.at[pl.ds(ck * CHUNK * HALF, CHUNK * HALF)],
                                 msg[ls], sem_l[ls])

        def wait_loads(j, ls, rs):
            @pl.when(valid(j))
            def _():
                ck = base + j
                off = ck * CHUNK
                pltpu.make_async_copy(snd.at[pl.ds(off, CHUNK)], idx_s[ls], sem_l[ls]).wait()
                pltpu.make_async_copy(rcv.at[pl.ds(off, CHUNK)], idx_r[rs], sem_l[ls]).wait()
                pltpu.make_async_copy(ep_h.at[pl.ds(0, CHUNK * HALF)], msg[ls], sem_l[ls]).wait()

        def fire_gather(j, ls):
            @pl.when(valid(j))
            def _():
                pltpu.async_copy(np_h.at[idx_s[ls]], gat[ls], sem_g[ls])

        def wait_scatter(j, ls, rs):
            @pl.when(valid(j))
            def _():
                pltpu.make_async_copy(sct[ls], seg_sh.at[idx_r[rs]], sem_s[ls]).wait()

        def process(j, ls, rs):
            wait_scatter(j - 2, ls, (rs + 2) % 4)

            @pl.when(valid(j))
            def _():
                pltpu.make_async_copy(np_h.at[idx_s[ls]], gat[ls], sem_g[ls]).wait()

                def row_body(r, rc):
                    for e4 in range(PACK):
                        e = r * PACK + e4
                        for kk in range(HALF // 16):
                            src = pl.ds(r * 128 + e4 * HALF + kk * 16, 16)
                            dst = pl.ds(kk * 16, 16)
                            sct[ls][e, dst] = jnp.maximum(
                                msg[ls][src] + gat[ls][e, dst], 0.0)
                    return rc

                lax.fori_loop(0, CROWS, row_body, 0)
                pltpu.async_copy(sct[ls], seg_sh.at[idx_r[rs]], sem_s[ls], add=True)

        fire_loads(0, 0, 0)
        fire_loads(1, 1, 1)
        wait_loads(0, 0, 0)
        fire_gather(0, 0)

        def loop(jo, carry):
            for b in range(4):
                j = jo * 4 + b
                wait_loads(j + 1, (b + 1) % 2, (b + 1) % 4)
                fire_gather(j + 1, (b + 1) % 2)
                process(j, b % 2, b)
                fire_loads(j + 2, b % 2, (b + 2) % 4)
            return carry

        NJ4 = (NJT + 3) // 4
        lax.fori_loop(0, NJ4, loop, 0)
        wait_scatter(4 * NJ4 - 2, 0, 2)
        wait_scatter(4 * NJ4 - 1, 1, 3)
        plsc.subcore_barrier()
        pltpu.sync_copy(seg_sh.at[rows], seg_o.at[rows])

    def do_cnt_pass(cnt_o, ck_base):
        pltpu.sync_copy(zh.at[rows], seg_sh.at[rows])
        pltpu.sync_copy(ones_h, ones_v)
        plsc.subcore_barrier()
        HJT = (HCHUNKS + NS - 1) // NS
        cbase = ck_base + s * HJT

        def cvalid(j):
            return jnp.logical_and(j < HJT, cbase + j < ck_base + HCHUNKS)

        def fire(j, b):
            @pl.when(cvalid(j))
            def _():
                off = (cbase + j) * CHUNK
                pltpu.async_copy(rcv.at[pl.ds(off, CHUNK)], idx_r[b], sem_l[b])

        def cproc(j, b):
            @pl.when(cvalid(j))
            def _():
                off = (cbase + j) * CHUNK
                pltpu.make_async_copy(rcv.at[pl.ds(off, CHUNK)], idx_r[b], sem_l[b]).wait()
                pltpu.sync_copy(ones_v, seg_sh.at[idx_r[b]], add=True)

        fire(0, 0)
        fire(1, 1)

        def cloop(jo, carry):
            for b in range(2):
                j = jo * 2 + b
                cproc(j, b)
                fire(j + 2, b)
            return carry

        lax.fori_loop(0, (HJT + 1) // 2, cloop, 0)
        plsc.subcore_barrier()
        pltpu.sync_copy(seg_sh.at[rows], cnt_o.at[rows])

    @pl.when(c == 0)
    def _():
        do_pass(ep0, np0, seg0_o)
        do_cnt_pass(cnta_o, 0)

    @pl.when(c == 1)
    def _():
        do_pass(ep1, np1, seg1_o)
        do_cnt_pass(cntb_o, HCHUNKS)


def _sc_scatter(eps, nps, senders, receivers, zh, ones32):
    mesh = plsc.VectorSubcoreMesh(
        core_axis_name="c", subcore_axis_name="s", num_cores=2, num_subcores=NS)
    return pl.kernel(
        _sc_body,
        out_type=tuple([jax.ShapeDtypeStruct((N_PAD, HALF), f32)] * 4),
        mesh=mesh,
        compiler_params=pltpu.CompilerParams(use_tc_tiling_on_sc=False),
        scratch_types=[
            pltpu.VMEM((CHUNK,), i32),
            pltpu.VMEM((CHUNK,), i32),
            pltpu.VMEM((CHUNK,), i32),
            pltpu.VMEM((CHUNK,), i32),
            pltpu.VMEM((CHUNK,), i32),
            pltpu.VMEM((CHUNK,), i32),
            pltpu.VMEM((CHUNK * HALF,), f32),
            pltpu.VMEM((CHUNK * HALF,), f32),
            pltpu.VMEM((CHUNK, HALF), f32),
            pltpu.VMEM((CHUNK, HALF), f32),
            pltpu.VMEM((CHUNK, HALF), f32),
            pltpu.VMEM((CHUNK, HALF), f32),
            pltpu.VMEM((CHUNK, HALF), f32),
            pltpu.VMEM_SHARED((N_PAD, HALF), f32),
            pltpu.SemaphoreType.DMA,
            pltpu.SemaphoreType.DMA,
            pltpu.SemaphoreType.DMA,
            pltpu.SemaphoreType.DMA,
            pltpu.SemaphoreType.DMA,
            pltpu.SemaphoreType.DMA,
        ],
    )(*eps, *nps, senders, receivers, zh, ones32)


def _update_body(seg0, seg1, cnta, cntb, n, bg, Wnn, Wni, Wng, bn_, Wge,
                 Wgn, Wgg, bg_, Wrn, brn, Wrg, brg, nout_o, gout_o, acc_e, acc_n):
    i = pl.program_id(0)
    nblocks = pl.num_programs(0)

    @pl.when(i == 0)
    def _():
        acc_e[...] = jnp.zeros_like(acc_e)
        acc_n[...] = jnp.zeros_like(acc_n)

    seg = jnp.concatenate([seg0[...], seg1[...]], axis=1)
    deg = jnp.maximum(cnta[:, 0:1] + cntb[:, 0:1], 1.0)
    inc = seg / deg
    u = jnp.maximum(bg[...], 0.0)
    cn = jnp.dot(u, Wng[...], preferred_element_type=f32) + bn_[...]
    n2 = jnp.maximum(
        jnp.dot(n[...], Wnn[...], preferred_element_type=f32)
        + jnp.dot(inc, Wni[...], preferred_element_type=f32) + cn, 0.0)
    nout_o[...] = _sigmoid(jnp.dot(n2, Wrn[...], preferred_element_type=f32) + brn[...])
    acc_e[...] += jnp.sum(seg, axis=0, keepdims=True)
    acc_n[...] += jnp.sum(n2, axis=0, keepdims=True)

    @pl.when(i == nblocks - 1)
    def _():
        mean_e2 = acc_e[...] * (1.0 / N_EDGES)
        mean_n2 = acc_n[...] * (1.0 / N_NODES)
        u2 = jnp.maximum(
            jnp.dot(mean_e2, Wge[...], preferred_element_type=f32)
            + jnp.dot(mean_n2, Wgn[...], preferred_element_type=f32)
            + jnp.dot(u, Wgg[...], preferred_element_type=f32) + bg_[...], 0.0)
        gout_o[...] = _sigmoid(jnp.dot(u2, Wrg[...], preferred_element_type=f32) + brg[...])


def _update(seg0, seg1, cnta, cntb, n, bg, Wn_n, Wn_i, Wn_g, bn_, Wg_e, Wg_n,
            Wg_g, bg_, Wr_n, br_n, Wr_g, br_g):
    grid = (N_NODES // NENC_BLK,)
    return pl.pallas_call(
        _update_body,
        grid=grid,
        in_specs=[pl.BlockSpec((NENC_BLK, HALF), lambda i: (i, 0))] * 4 + [
            pl.BlockSpec((NENC_BLK, 16), lambda i: (i, 0)),
            _full((1, 4)), _full((16, 128)), _full((64, 128)), _full((4, 128)),
            _full((1, 128)), _full((64, 32)), _full((128, 32)), _full((4, 32)),
            _full((1, 32)), _full((128, 1)), _full((1, 1)), _full((32, 1)),
            _full((1, 1)),
        ],
        out_specs=[
            pl.BlockSpec((NENC_BLK, 1), lambda i: (i, 0)),
            pl.BlockSpec((1, 1), lambda i: (0, 0)),
        ],
        out_shape=[
            jax.ShapeDtypeStruct((N_NODES, 1), f32),
            jax.ShapeDtypeStruct((1, 1), f32),
        ],
        scratch_shapes=[
            pltpu.VMEM((1, 64), f32),
            pltpu.VMEM((1, 128), f32),
        ],
    )(seg0, seg1, cnta, cntb, n, bg, Wn_n, Wn_i, Wn_g, bn_, Wg_e, Wg_n,
      Wg_g, bg_, Wr_n, br_n, Wr_g, br_g)


def kernel(x, edge_attr, senders, receivers, We1, be1, We2, be2, Wn1, bn1,
           Wn2, bn2, bg_enc, We_e, We_s, We_g, be, Wn_n, Wn_i, Wn_g, bn,
           Wg_e, Wg_n, Wg_g, bg, Wr_n, br_n, Wr_g, br_g):
    r = lambda v: v.reshape(1, -1)
    eps = _edge_encode(edge_attr, We1, be1, We2, be2, We_e, We_g, be, bg_enc)
    n_enc, np0, np1 = _node_encode(x, Wn1, r(bn1), Wn2, r(bn2), We_s)
    zh = jnp.zeros((N_PAD, HALF), f32)
    ones32 = jnp.ones((CHUNK, HALF), f32)
    eps_flat = tuple(e.reshape(-1) for e in eps)
    seg0, seg1, cnta, cntb = _sc_scatter(eps_flat, (np0, np1), senders,
                                         receivers, zh, ones32)
    node_out, global_out = _update(seg0, seg1, cnta, cntb, n_enc, r(bg_enc),
                                   Wn_n, Wn_i, Wn_g, r(bn), Wg_e, Wg_n, Wg_g,
                                   r(bg), Wr_n, r(br_n), Wr_g, r(br_g))
    return node_out, global_out
